# Initial kernel scaffold; baseline (speedup 1.0000x reference)
import jax
import jax.numpy as jnp
from jax.experimental import pallas as pl


def kernel(x, edge_index, lin1_W, lin1_b, conv1_W, conv1_b, conv2_W, conv2_b, pool_Wrel, pool_brel, pool_Wroot, conv3_W, conv3_b, conv4_W, conv4_b, conv5_W, conv5_b, conv6_W, conv6_b, lin2_W, lin2_b):
    raise NotImplementedError("write your pallas kernel here")


# speedup vs baseline: 1.0592x; 1.0592x over previous
import functools

import jax
import jax.numpy as jnp
from jax.experimental import pallas as pl
from jax.experimental.pallas import tpu as pltpu

N = 10000
E = 320000
TMAX = 128
NT = 128
K = 5000


def _leaky(x):
    return jnp.where(x > 0, x, 0.01 * x)


def _lin1_body(x_ref, w_ref, b_ref, o_ref):
    o_ref[...] = _leaky(
        jnp.dot(x_ref[...], w_ref[...], preferred_element_type=jnp.float32)
        + b_ref[...]
    )


def _lin1(x, W, b):
    n_pad = 10240
    xp = jnp.zeros((n_pad, TMAX), jnp.float32).at[:N].set(x)
    out = pl.pallas_call(
        _lin1_body,
        grid=(n_pad // 512,),
        in_specs=[
            pl.BlockSpec((512, TMAX), lambda i: (i, 0)),
            pl.BlockSpec((TMAX, NT), lambda i: (0, 0)),
            pl.BlockSpec((1, NT), lambda i: (0, 0)),
        ],
        out_specs=pl.BlockSpec((512, NT), lambda i: (i, 0)),
        out_shape=jax.ShapeDtypeStruct((n_pad, NT), jnp.float32),
    )(xp, W, b.reshape(1, NT))
    return out[:N]


def _gcn(x, W, b, row, col, mask, n):
    xw = x @ W
    deg = jax.ops.segment_sum(mask, col, num_segments=n) + 1.0
    dinv = 1.0 / jnp.sqrt(deg)
    norm = mask * dinv[row] * dinv[col]
    out = jax.ops.segment_sum(norm[:, None] * xw[row], col, num_segments=n)
    out = out + (dinv * dinv)[:, None] * xw
    return out + b


def kernel(x, edge_index, lin1_W, lin1_b, conv1_W, conv1_b, conv2_W, conv2_b,
           pool_Wrel, pool_brel, pool_Wroot, conv3_W, conv3_b, conv4_W, conv4_b,
           conv5_W, conv5_b, conv6_W, conv6_b, lin2_W, lin2_b):
    row = edge_index[0]
    col = edge_index[1]
    mask = jnp.ones((E,), jnp.float32)

    h = _lin1(x, lin1_W, lin1_b)
    h = _leaky(_gcn(h, conv1_W, conv1_b, row, col, mask, N))
    h = _leaky(_gcn(h, conv2_W, conv2_b, row, col, mask, N))

    aggr = jax.ops.segment_sum(mask[:, None] * h[row], col, num_segments=N)
    score = (aggr @ pool_Wrel + pool_brel + h @ pool_Wroot).reshape(-1)
    s = jnp.tanh(score)
    vals, perm = jax.lax.top_k(s, K)
    h = h[perm] * vals[:, None]

    inv = jnp.full((N,), -1, jnp.int32).at[perm].set(jnp.arange(K, dtype=jnp.int32))
    nr = inv[row]
    nc = inv[col]
    m2 = ((nr >= 0) & (nc >= 0)).astype(jnp.float32)
    r2 = jnp.maximum(nr, 0)
    c2 = jnp.maximum(nc, 0)

    h = _leaky(_gcn(h, conv3_W, conv3_b, r2, c2, m2, K))
    h = _leaky(_gcn(h, conv4_W, conv4_b, r2, c2, m2, K))
    h = _leaky(_gcn(h, conv5_W, conv5_b, r2, c2, m2, K))
    h = _leaky(_gcn(h, conv6_W, conv6_b, r2, c2, m2, K))
    h = h @ lin2_W + lin2_b
    return jax.nn.relu(h)


# speedup vs baseline: 15.6684x; 14.7931x over previous
import functools

import jax
import jax.numpy as jnp
from jax import lax
from jax.experimental import pallas as pl
from jax.experimental.pallas import tpu as pltpu
from jax.experimental.pallas import tpu_sc as plsc

N = 10000
E = 320000
D = 128
K = 5000

NC = 2
NS = 16
NW = NC * NS

N_PAD = 10240
EPW = 10240
E_PAD = NW * EPW
IDXR = EPW // 128

K_PAD = 5120
P_TRASH = 1024
K_ACC = K_PAD + P_TRASH

_MESH = plsc.VectorSubcoreMesh(core_axis_name="c", subcore_axis_name="s")


def _leaky(v):
    return jnp.where(v > 0, v, 0.01 * v)


def _sc_degree(cidx3, zeros1):
    @functools.partial(
        pl.kernel,
        out_type=jax.ShapeDtypeStruct((NC, N_PAD), jnp.float32),
        mesh=_MESH,
        scratch_types=[
            pltpu.VMEM((IDXR, 128), jnp.int32),
            pltpu.VMEM((1, 128), jnp.float32),
            pltpu.VMEM_SHARED((N_PAD,), jnp.float32),
            pltpu.SemaphoreType.DMA,
        ],
    )
    def body(cidx_hbm, zeros_hbm, out_hbm, cidx_v, ones_v, hist, sem):
        c = lax.axis_index("c")
        s = lax.axis_index("s")
        wid = c * NS + s
        rows_per_sub = N_PAD // NS
        pltpu.sync_copy(zeros_hbm.at[pl.ds(s * rows_per_sub, rows_per_sub)],
                        hist.at[pl.ds(s * rows_per_sub, rows_per_sub)])
        pltpu.sync_copy(cidx_hbm.at[wid], cidx_v)
        for j in range(8):
            ones_v[0, pl.ds(j * 16, 16)] = jnp.ones((16,), jnp.float32)
        plsc.subcore_barrier()

        def step(k, carry):
            pltpu.sync_copy(ones_v.at[0], hist.at[cidx_v.at[k]], add=True)
            return carry

        lax.fori_loop(0, IDXR, step, 0)
        plsc.subcore_barrier()
        pltpu.sync_copy(hist.at[pl.ds(s * rows_per_sub, rows_per_sub)],
                        out_hbm.at[c, pl.ds(s * rows_per_sub, rows_per_sub)])

    return body(cidx3, zeros1)


def _sc_segsum(y, rcidx4, zeros2, n_acc):
    @functools.partial(
        pl.kernel,
        out_type=jax.ShapeDtypeStruct((NC, n_acc, D), jnp.float32),
        mesh=_MESH,
        scratch_types=[
            pltpu.VMEM((2, 128), jnp.int32),
            pltpu.VMEM((2, 128), jnp.int32),
            pltpu.VMEM((128, D), jnp.float32),
            pltpu.VMEM((128, D), jnp.float32),
            pltpu.VMEM_SHARED((n_acc, D), jnp.float32),
            pltpu.SemaphoreType.DMA,
            pltpu.SemaphoreType.DMA,
            pltpu.SemaphoreType.DMA,
            pltpu.SemaphoreType.DMA,
        ],
    )
    def body(y_hbm, rc_hbm, zeros_hbm, out_hbm,
             idx0, idx1, buf0, buf1, acc, isem0, isem1, gsem0, gsem1):
        c = lax.axis_index("c")
        s = lax.axis_index("s")
        wid = c * NS + s
        rows_per_sub = n_acc // NS
        pltpu.sync_copy(zeros_hbm.at[pl.ds(s * rows_per_sub, rows_per_sub)],
                        acc.at[pl.ds(s * rows_per_sub, rows_per_sub)])
        plsc.subcore_barrier()

        pltpu.sync_copy(rc_hbm.at[wid, 0], idx0)
        pltpu.async_copy(y_hbm.at[idx0.at[0]], buf0, gsem0)
        pltpu.async_copy(rc_hbm.at[wid, 1], idx1, isem1)

        def step(kk, carry):
            k0 = 2 * kk
            pltpu.make_async_copy(rc_hbm.at[wid, k0 + 1], idx1, isem1).wait()
            pltpu.async_copy(y_hbm.at[idx1.at[0]], buf1, gsem1)

            pltpu.make_async_copy(y_hbm.at[idx0.at[0]], buf0, gsem0).wait()
            pltpu.sync_copy(buf0, acc.at[idx0.at[1]], add=True)

            @pl.when(k0 + 2 < IDXR)
            def _():
                pltpu.async_copy(rc_hbm.at[wid, k0 + 2], idx0, isem0)

            pltpu.make_async_copy(y_hbm.at[idx1.at[0]], buf1, gsem1).wait()
            pltpu.sync_copy(buf1, acc.at[idx1.at[1]], add=True)

            @pl.when(k0 + 3 < IDXR)
            def _():
                pltpu.async_copy(rc_hbm.at[wid, k0 + 3], idx1, isem1)

            @pl.when(k0 + 2 < IDXR)
            def _():
                pltpu.make_async_copy(rc_hbm.at[wid, k0 + 2], idx0, isem0).wait()
                pltpu.async_copy(y_hbm.at[idx0.at[0]], buf0, gsem0)

            return carry

        lax.fori_loop(0, IDXR // 2, step, 0)
        plsc.subcore_barrier()
        pltpu.sync_copy(acc.at[pl.ds(s * rows_per_sub, rows_per_sub)],
                        out_hbm.at[c, pl.ds(s * rows_per_sub, rows_per_sub)])

    return body(y, rcidx4, zeros2)


def _sc_remap(ridx3, cidx3, rank1, zerosk):
    @functools.partial(
        pl.kernel,
        out_type=(
            jax.ShapeDtypeStruct((NW, IDXR, 128), jnp.int32),
            jax.ShapeDtypeStruct((NW, IDXR, 128), jnp.int32),
            jax.ShapeDtypeStruct((NC, K_ACC), jnp.float32),
        ),
        mesh=_MESH,
        scratch_types=[
            pltpu.VMEM((IDXR, 128), jnp.int32),
            pltpu.VMEM((IDXR, 128), jnp.int32),
            pltpu.VMEM((IDXR, 128), jnp.int32),
            pltpu.VMEM((IDXR, 128), jnp.int32),
            pltpu.VMEM((1, 128), jnp.int32),
            pltpu.VMEM((1, 128), jnp.int32),
            pltpu.VMEM((1, 128), jnp.float32),
            pltpu.VMEM_SHARED((K_ACC,), jnp.float32),
            pltpu.SemaphoreType.DMA,
        ],
    )
    def body(ridx_hbm, cidx_hbm, rank_hbm, zeros_hbm, r2_hbm, c2_hbm, hist_hbm,
             ridx_v, cidx_v, r2_v, c2_v, rr_v, rc_v, ones_v, hist, sem):
        c = lax.axis_index("c")
        s = lax.axis_index("s")
        wid = c * NS + s
        rows_per_sub = K_ACC // NS
        pltpu.sync_copy(zeros_hbm.at[pl.ds(s * rows_per_sub, rows_per_sub)],
                        hist.at[pl.ds(s * rows_per_sub, rows_per_sub)])
        pltpu.sync_copy(ridx_hbm.at[wid], ridx_v)
        pltpu.sync_copy(cidx_hbm.at[wid], cidx_v)
        for j in range(8):
            ones_v[0, pl.ds(j * 16, 16)] = jnp.ones((16,), jnp.float32)
        plsc.subcore_barrier()

        def step(k, carry):
            pltpu.async_copy(rank_hbm.at[ridx_v.at[k]], rr_v.at[0], sem).wait()
            pltpu.async_copy(rank_hbm.at[cidx_v.at[k]], rc_v.at[0], sem).wait()
            ebase = wid * EPW + k * 128
            for j in range(8):
                a = rr_v[0, pl.ds(j * 16, 16)]
                b = rc_v[0, pl.ds(j * 16, 16)]
                live = (a < K) & (b < K)
                eid = ebase + j * 16 + lax.iota(jnp.int32, 16)
                r2 = jnp.where(live, a, eid & 4095)
                c2 = jnp.where(live, b, K_PAD + (eid & (P_TRASH - 1)))
                r2_v[k, pl.ds(j * 16, 16)] = r2
                c2_v[k, pl.ds(j * 16, 16)] = c2
            pltpu.sync_copy(ones_v.at[0], hist.at[c2_v.at[k]], add=True)
            return carry

        lax.fori_loop(0, IDXR, step, 0)
        pltpu.sync_copy(r2_v, r2_hbm.at[wid])
        pltpu.sync_copy(c2_v, c2_hbm.at[wid])
        plsc.subcore_barrier()
        pltpu.sync_copy(hist.at[pl.ds(s * rows_per_sub, rows_per_sub)],
                        hist_hbm.at[c, pl.ds(s * rows_per_sub, rows_per_sub)])

    return body(ridx3, cidx3, rank1, zerosk)


def _sc_scatter_rows(z, rank2):
    n_chunks = N_PAD // 128

    @functools.partial(
        pl.kernel,
        out_type=jax.ShapeDtypeStruct((K_ACC, D), jnp.float32),
        mesh=_MESH,
        scratch_types=[
            pltpu.VMEM((128, D), jnp.float32),
            pltpu.VMEM((1, 128), jnp.int32),
            pltpu.VMEM((1, 128), jnp.int32),
            pltpu.SemaphoreType.DMA,
        ],
    )
    def body(z_hbm, rank_hbm, out_hbm, rows_v, rk_v, idx_v, sem):
        c = lax.axis_index("c")
        s = lax.axis_index("s")
        wid = c * NS + s

        def step(t, carry):
            cid = wid + NW * t

            @pl.when(cid < n_chunks)
            def _():
                pltpu.sync_copy(z_hbm.at[pl.ds(cid * 128, 128)], rows_v)
                pltpu.sync_copy(rank_hbm.at[cid], rk_v.at[0])
                for j in range(8):
                    rk = rk_v[0, pl.ds(j * 16, 16)]
                    nid = cid * 128 + j * 16 + lax.iota(jnp.int32, 16)
                    idx = jnp.where(rk < K, rk, K_PAD + (nid & (P_TRASH - 1)))
                    idx_v[0, pl.ds(j * 16, 16)] = idx
                pltpu.async_copy(rows_v, out_hbm.at[idx_v.at[0]], sem).wait()

            return carry

        lax.fori_loop(0, (n_chunks + NW - 1) // NW, step, 0)

    return body(z, rank2)


_BLK = 512


def _tc_lin1(x, W, b):
    def body(x_ref, w_ref, b_ref, o_ref):
        o_ref[...] = _leaky(
            jnp.dot(x_ref[...], w_ref[...], preferred_element_type=jnp.float32)
            + b_ref[...])

    return pl.pallas_call(
        body,
        grid=(N_PAD // _BLK,),
        in_specs=[
            pl.BlockSpec((_BLK, D), lambda i: (i, 0)),
            pl.BlockSpec((D, D), lambda i: (0, 0)),
            pl.BlockSpec((1, D), lambda i: (0, 0)),
        ],
        out_specs=pl.BlockSpec((_BLK, D), lambda i: (i, 0)),
        out_shape=jax.ShapeDtypeStruct((N_PAD, D), jnp.float32),
    )(x, W, b.reshape(1, D))


def _tc_prep(h, W, d0, d1):
    n = h.shape[0]

    def body(h_ref, w_ref, d0_ref, d1_ref, y_ref, di_ref):
        dinv = lax.rsqrt(d0_ref[...] + d1_ref[...] + 1.0)
        y_ref[...] = dinv * jnp.dot(h_ref[...], w_ref[...],
                                    preferred_element_type=jnp.float32)
        di_ref[...] = dinv

    return pl.pallas_call(
        body,
        grid=(n // _BLK,),
        in_specs=[
            pl.BlockSpec((_BLK, D), lambda i: (i, 0)),
            pl.BlockSpec((D, D), lambda i: (0, 0)),
            pl.BlockSpec((_BLK, 1), lambda i: (i, 0)),
            pl.BlockSpec((_BLK, 1), lambda i: (i, 0)),
        ],
        out_specs=[
            pl.BlockSpec((_BLK, D), lambda i: (i, 0)),
            pl.BlockSpec((_BLK, 1), lambda i: (i, 0)),
        ],
        out_shape=[
            jax.ShapeDtypeStruct((n, D), jnp.float32),
            jax.ShapeDtypeStruct((n, 1), jnp.float32),
        ],
    )(h, W, d0.reshape(n, 1), d1.reshape(n, 1))


def _tc_mid(p0, p1, y_prev, dinv, b_prev, W, scale_out):
    n = y_prev.shape[0]

    def body(p0_ref, p1_ref, y_ref, di_ref, b_ref, w_ref, so_ref, o_ref):
        h = _leaky(di_ref[...] * (p0_ref[...] + p1_ref[...] + y_ref[...])
                   + b_ref[...])
        o_ref[...] = so_ref[...] * jnp.dot(h, w_ref[...],
                                           preferred_element_type=jnp.float32)

    return pl.pallas_call(
        body,
        grid=(n // _BLK,),
        in_specs=[
            pl.BlockSpec((_BLK, D), lambda i: (i, 0)),
            pl.BlockSpec((_BLK, D), lambda i: (i, 0)),
            pl.BlockSpec((_BLK, D), lambda i: (i, 0)),
            pl.BlockSpec((_BLK, 1), lambda i: (i, 0)),
            pl.BlockSpec((1, D), lambda i: (0, 0)),
            pl.BlockSpec((D, D), lambda i: (0, 0)),
            pl.BlockSpec((_BLK, 1), lambda i: (i, 0)),
        ],
        out_specs=pl.BlockSpec((_BLK, D), lambda i: (i, 0)),
        out_shape=jax.ShapeDtypeStruct((n, D), jnp.float32),
    )(p0, p1, y_prev, dinv, b_prev.reshape(1, D), W, scale_out)


def _tc_post(p0, p1, y_prev, dinv, b_prev):
    n = y_prev.shape[0]

    def body(p0_ref, p1_ref, y_ref, di_ref, b_ref, o_ref):
        o_ref[...] = _leaky(
            di_ref[...] * (p0_ref[...] + p1_ref[...] + y_ref[...]) + b_ref[...])

    return pl.pallas_call(
        body,
        grid=(n // _BLK,),
        in_specs=[
            pl.BlockSpec((_BLK, D), lambda i: (i, 0)),
            pl.BlockSpec((_BLK, D), lambda i: (i, 0)),
            pl.BlockSpec((_BLK, D), lambda i: (i, 0)),
            pl.BlockSpec((_BLK, 1), lambda i: (i, 0)),
            pl.BlockSpec((1, D), lambda i: (0, 0)),
        ],
        out_specs=pl.BlockSpec((_BLK, D), lambda i: (i, 0)),
        out_shape=jax.ShapeDtypeStruct((n, D), jnp.float32),
    )(p0, p1, y_prev, dinv, b_prev.reshape(1, D))


def _tc_score(pp0, pp1, h2, Wrel_p, Wroot_p, brel):
    def body(p0_ref, p1_ref, h_ref, wr_ref, wo_ref, b_ref, o_ref):
        aggr = p0_ref[...] + p1_ref[...]
        sc = (jnp.dot(aggr, wr_ref[...], preferred_element_type=jnp.float32)
              + jnp.dot(h_ref[...], wo_ref[...], preferred_element_type=jnp.float32)
              + b_ref[...])
        o_ref[...] = jnp.tanh(sc)

    return pl.pallas_call(
        body,
        grid=(N_PAD // _BLK,),
        in_specs=[
            pl.BlockSpec((_BLK, D), lambda i: (i, 0)),
            pl.BlockSpec((_BLK, D), lambda i: (i, 0)),
            pl.BlockSpec((_BLK, D), lambda i: (i, 0)),
            pl.BlockSpec((D, D), lambda i: (0, 0)),
            pl.BlockSpec((D, D), lambda i: (0, 0)),
            pl.BlockSpec((1, 1), lambda i: (0, 0)),
        ],
        out_specs=pl.BlockSpec((_BLK, D), lambda i: (i, 0)),
        out_shape=jax.ShapeDtypeStruct((N_PAD, D), jnp.float32),
    )(pp0, pp1, h2, Wrel_p, Wroot_p, brel.reshape(1, 1))


_JBLK = 2048


def _tc_rank(s_col, s_row):
    def body(sc_ref, sr_ref, o_ref):
        i = pl.program_id(0)
        j = pl.program_id(1)
        si = sc_ref[...]
        sj = sr_ref[...]
        ii = (lax.broadcasted_iota(jnp.int32, (_BLK, _JBLK), 0) + i * _BLK)
        jj = (lax.broadcasted_iota(jnp.int32, (_BLK, _JBLK), 1) + j * _JBLK)
        si_e = jnp.where(ii < N, si, -2.0)
        sj_e = jnp.where(jj < N, sj, -2.0)
        t = jnp.where(sj_e > si_e, 1, 0) + jnp.where(
            (sj_e == si_e) & (jj < ii), 1, 0)
        part = jnp.sum(t, axis=1, keepdims=True)

        @pl.when(j == 0)
        def _():
            o_ref[...] = jnp.zeros_like(o_ref)

        o_ref[...] += part

    return pl.pallas_call(
        body,
        grid=(N_PAD // _BLK, N_PAD // _JBLK),
        in_specs=[
            pl.BlockSpec((_BLK, 1), lambda i, j: (i, 0)),
            pl.BlockSpec((1, _JBLK), lambda i, j: (0, j)),
        ],
        out_specs=pl.BlockSpec((_BLK, 1), lambda i, j: (i, 0)),
        out_shape=jax.ShapeDtypeStruct((N_PAD, 1), jnp.int32),
    )(s_col, s_row)


def _tc_zmul(h2, s_col):
    def body(h_ref, s_ref, o_ref):
        o_ref[...] = h_ref[...] * s_ref[...]

    return pl.pallas_call(
        body,
        grid=(N_PAD // _BLK,),
        in_specs=[
            pl.BlockSpec((_BLK, D), lambda i: (i, 0)),
            pl.BlockSpec((_BLK, 1), lambda i: (i, 0)),
        ],
        out_specs=pl.BlockSpec((_BLK, D), lambda i: (i, 0)),
        out_shape=jax.ShapeDtypeStruct((N_PAD, D), jnp.float32),
    )(h2, s_col)


def _tc_final(p0, p1, y_prev, dinv, b_prev, W2, b2):
    n = y_prev.shape[0]

    def body(p0_ref, p1_ref, y_ref, di_ref, b_ref, w_ref, b2_ref, o_ref):
        h = _leaky(di_ref[...] * (p0_ref[...] + p1_ref[...] + y_ref[...])
                   + b_ref[...])
        o_ref[...] = jnp.maximum(
            jnp.dot(h, w_ref[...], preferred_element_type=jnp.float32)
            + b2_ref[...], 0.0)

    return pl.pallas_call(
        body,
        grid=(n // _BLK,),
        in_specs=[
            pl.BlockSpec((_BLK, D), lambda i: (i, 0)),
            pl.BlockSpec((_BLK, D), lambda i: (i, 0)),
            pl.BlockSpec((_BLK, D), lambda i: (i, 0)),
            pl.BlockSpec((_BLK, 1), lambda i: (i, 0)),
            pl.BlockSpec((1, D), lambda i: (0, 0)),
            pl.BlockSpec((D, D), lambda i: (0, 0)),
            pl.BlockSpec((1, D), lambda i: (0, 0)),
        ],
        out_specs=pl.BlockSpec((_BLK, D), lambda i: (i, 0)),
        out_shape=jax.ShapeDtypeStruct((n, D), jnp.float32),
    )(p0, p1, y_prev, dinv, b_prev.reshape(1, D), W2, b2.reshape(1, D))


def kernel(x, edge_index, lin1_W, lin1_b, conv1_W, conv1_b, conv2_W, conv2_b,
           pool_Wrel, pool_brel, pool_Wroot, conv3_W, conv3_b, conv4_W, conv4_b,
           conv5_W, conv5_b, conv6_W, conv6_b, lin2_W, lin2_b):
    row = edge_index[0]
    col = edge_index[1]

    epad = E_PAD - E
    pad_ids = jnp.arange(epad, dtype=jnp.int32)
    row_p = jnp.concatenate([row, pad_ids % N])
    col_p = jnp.concatenate([col, N + (pad_ids % (N_PAD - N))])
    ridx3 = row_p.reshape(NW, IDXR, 128)
    cidx3 = col_p.reshape(NW, IDXR, 128)
    rcidx4 = jnp.stack([ridx3, cidx3], axis=2)

    zeros2 = jnp.zeros((N_PAD, D), jnp.float32)
    zeros1 = jnp.zeros((N_PAD,), jnp.float32)
    zeros1k = jnp.zeros((K_ACC,), jnp.float32)

    xp = jnp.zeros((N_PAD, D), jnp.float32).at[:N].set(x)

    degp = _sc_degree(cidx3, zeros1)

    h1 = _tc_lin1(xp, lin1_W, lin1_b)

    y1, dinv1 = _tc_prep(h1, conv1_W, degp[0], degp[1])
    P1 = _sc_segsum(y1, rcidx4, zeros2, N_PAD)
    y2 = _tc_mid(P1[0], P1[1], y1, dinv1, conv1_b, conv2_W, dinv1)
    P2 = _sc_segsum(y2, rcidx4, zeros2, N_PAD)
    h2 = _tc_post(P2[0], P2[1], y2, dinv1, conv2_b)

    P3 = _sc_segsum(h2, rcidx4, zeros2, N_PAD)
    Wrel_p = jnp.zeros((D, D), jnp.float32).at[:, 0:1].set(pool_Wrel)
    Wroot_p = jnp.zeros((D, D), jnp.float32).at[:, 0:1].set(pool_Wroot)
    s_full = _tc_score(P3[0], P3[1], h2, Wrel_p, Wroot_p, pool_brel)
    s_col = s_full[:, 0:1]
    s_row = s_col.reshape(1, N_PAD)
    rank = _tc_rank(s_col, s_row)
    rank1 = rank.reshape(N_PAD)
    rank2 = rank.reshape(IDXR, 128)

    z = _tc_zmul(h2, s_col)
    r2_3, c2_3, hist2p = _sc_remap(ridx3, cidx3, rank1, zeros1k)
    rcidx4_2 = jnp.stack([r2_3, c2_3], axis=2)
    h_pool = _sc_scatter_rows(z, rank2)

    def padN(y):
        return jnp.zeros((N_PAD, D), jnp.float32).at[:K_PAD].set(y)

    hp = h_pool[:K_PAD]
    d2a = hist2p[0][:K_PAD]
    d2b = hist2p[1][:K_PAD]
    y3, dinv2 = _tc_prep(hp, conv3_W, d2a, d2b)
    P4 = _sc_segsum(padN(y3), rcidx4_2, zeros2, N_PAD)
    y4 = _tc_mid(P4[0][:K_PAD], P4[1][:K_PAD], y3, dinv2, conv3_b, conv4_W, dinv2)
    P5 = _sc_segsum(padN(y4), rcidx4_2, zeros2, N_PAD)
    y5 = _tc_mid(P5[0][:K_PAD], P5[1][:K_PAD], y4, dinv2, conv4_b, conv5_W, dinv2)
    P6 = _sc_segsum(padN(y5), rcidx4_2, zeros2, N_PAD)
    y6 = _tc_mid(P6[0][:K_PAD], P6[1][:K_PAD], y5, dinv2, conv5_b, conv6_W, dinv2)
    P7 = _sc_segsum(padN(y6), rcidx4_2, zeros2, N_PAD)
    out = _tc_final(P7[0][:K_PAD], P7[1][:K_PAD], y6, dinv2, conv6_b,
                    lin2_W, lin2_b)
    return out[:K]


# speedup vs baseline: 18.0473x; 1.1518x over previous
import functools

import jax
import jax.numpy as jnp
from jax import lax
from jax.experimental import pallas as pl
from jax.experimental.pallas import tpu as pltpu
from jax.experimental.pallas import tpu_sc as plsc

N = 10000
E = 320000
D = 128
K = 5000

NC = 2
NS = 16
NW = NC * NS

N_PAD = 10240
EPW = 10240
E_PAD = NW * EPW
IDXR = EPW // 128

K_PAD = 5120
P_TRASH = 1024
K_ACC = K_PAD + P_TRASH

_MESH = plsc.VectorSubcoreMesh(core_axis_name="c", subcore_axis_name="s")


def _leaky(v):
    return jnp.where(v > 0, v, 0.01 * v)


def _sc_degree(cidx3, zeros1):
    @functools.partial(
        pl.kernel,
        out_type=jax.ShapeDtypeStruct((NC, N_PAD), jnp.float32),
        mesh=_MESH,
        scratch_types=[
            pltpu.VMEM((IDXR, 128), jnp.int32),
            pltpu.VMEM((1, 128), jnp.float32),
            pltpu.VMEM_SHARED((N_PAD,), jnp.float32),
            pltpu.SemaphoreType.DMA,
        ],
    )
    def body(cidx_hbm, zeros_hbm, out_hbm, cidx_v, ones_v, hist, sem):
        c = lax.axis_index("c")
        s = lax.axis_index("s")
        wid = c * NS + s
        rows_per_sub = N_PAD // NS
        pltpu.sync_copy(zeros_hbm.at[pl.ds(s * rows_per_sub, rows_per_sub)],
                        hist.at[pl.ds(s * rows_per_sub, rows_per_sub)])
        pltpu.sync_copy(cidx_hbm.at[wid], cidx_v)
        for j in range(8):
            ones_v[0, pl.ds(j * 16, 16)] = jnp.ones((16,), jnp.float32)
        plsc.subcore_barrier()

        def step(k, carry):
            pltpu.sync_copy(ones_v.at[0], hist.at[cidx_v.at[k]], add=True)
            return carry

        lax.fori_loop(0, IDXR, step, 0)
        plsc.subcore_barrier()
        pltpu.sync_copy(hist.at[pl.ds(s * rows_per_sub, rows_per_sub)],
                        out_hbm.at[c, pl.ds(s * rows_per_sub, rows_per_sub)])

    return body(cidx3, zeros1)


def _sc_segsum(y, rcidx4, zeros2, n_acc):
    @functools.partial(
        pl.kernel,
        out_type=jax.ShapeDtypeStruct((NC, n_acc, D), jnp.float32),
        mesh=_MESH,
        scratch_types=[
            [pltpu.VMEM((2, 128), jnp.int32) for _ in range(4)],
            pltpu.VMEM((128, D), jnp.float32),
            pltpu.VMEM((128, D), jnp.float32),
            pltpu.VMEM_SHARED((n_acc, D), jnp.float32),
            [pltpu.SemaphoreType.DMA for _ in range(4)],
            [pltpu.SemaphoreType.DMA for _ in range(2)],
            [pltpu.SemaphoreType.DMA for _ in range(2)],
        ],
    )
    def body(y_hbm, rc_hbm, zeros_hbm, out_hbm,
             idx, buf0, buf1, acc, isem, gsem, ssem):
        c = lax.axis_index("c")
        s = lax.axis_index("s")
        wid = c * NS + s
        buf = (buf0, buf1)
        rows_per_sub = n_acc // NS
        pltpu.sync_copy(zeros_hbm.at[pl.ds(s * rows_per_sub, rows_per_sub)],
                        acc.at[pl.ds(s * rows_per_sub, rows_per_sub)])
        plsc.subcore_barrier()

        pltpu.async_copy(rc_hbm.at[wid, 0], idx[0], isem[0])
        pltpu.async_copy(rc_hbm.at[wid, 1], idx[1], isem[1])

        def phase(k, K, kk):
            b = K % 2
            pb = (K - 1) % 2
            pq = (K - 1) % 4
            nq = (K + 2) % 4
            pltpu.make_async_copy(rc_hbm.at[wid, k], idx[K], isem[K]).wait()

            @pl.when(k >= 2)
            def _():
                pltpu.make_async_copy(buf[b], acc.at[idx[nq].at[1]], ssem[b]).wait()

            pltpu.async_copy(y_hbm.at[idx[K].at[0]], buf[b], gsem[b])

            @pl.when(k >= 1)
            def _():
                pltpu.make_async_copy(y_hbm.at[idx[pq].at[0]], buf[pb],
                                      gsem[pb]).wait()
                pltpu.async_copy(buf[pb], acc.at[idx[pq].at[1]], ssem[pb],
                                 add=True)

            @pl.when(k + 2 < IDXR)
            def _():
                pltpu.async_copy(rc_hbm.at[wid, k + 2], idx[nq], isem[nq])

        def step(kk, carry):
            for K in range(4):
                phase(4 * kk + K, K, kk)
            return carry

        lax.fori_loop(0, IDXR // 4, step, 0)
        pltpu.make_async_copy(y_hbm.at[idx[3].at[0]], buf1, gsem[1]).wait()
        pltpu.sync_copy(buf1, acc.at[idx[3].at[1]], add=True)
        pltpu.make_async_copy(buf0, acc.at[idx[2].at[1]], ssem[0]).wait()
        plsc.subcore_barrier()
        pltpu.sync_copy(acc.at[pl.ds(s * rows_per_sub, rows_per_sub)],
                        out_hbm.at[c, pl.ds(s * rows_per_sub, rows_per_sub)])

    return body(y, rcidx4, zeros2)


def _sc_remap(ridx3, cidx3, rank1, zerosk):
    @functools.partial(
        pl.kernel,
        out_type=(
            jax.ShapeDtypeStruct((NW, IDXR, 128), jnp.int32),
            jax.ShapeDtypeStruct((NW, IDXR, 128), jnp.int32),
            jax.ShapeDtypeStruct((NC, K_ACC), jnp.float32),
        ),
        mesh=_MESH,
        scratch_types=[
            pltpu.VMEM((IDXR, 128), jnp.int32),
            pltpu.VMEM((IDXR, 128), jnp.int32),
            pltpu.VMEM((IDXR, 128), jnp.int32),
            pltpu.VMEM((IDXR, 128), jnp.int32),
            pltpu.VMEM((1, 128), jnp.int32),
            pltpu.VMEM((1, 128), jnp.int32),
            pltpu.VMEM((1, 128), jnp.float32),
            pltpu.VMEM_SHARED((K_ACC,), jnp.float32),
            pltpu.SemaphoreType.DMA,
        ],
    )
    def body(ridx_hbm, cidx_hbm, rank_hbm, zeros_hbm, r2_hbm, c2_hbm, hist_hbm,
             ridx_v, cidx_v, r2_v, c2_v, rr_v, rc_v, ones_v, hist, sem):
        c = lax.axis_index("c")
        s = lax.axis_index("s")
        wid = c * NS + s
        rows_per_sub = K_ACC // NS
        pltpu.sync_copy(zeros_hbm.at[pl.ds(s * rows_per_sub, rows_per_sub)],
                        hist.at[pl.ds(s * rows_per_sub, rows_per_sub)])
        pltpu.sync_copy(ridx_hbm.at[wid], ridx_v)
        pltpu.sync_copy(cidx_hbm.at[wid], cidx_v)
        for j in range(8):
            ones_v[0, pl.ds(j * 16, 16)] = jnp.ones((16,), jnp.float32)
        plsc.subcore_barrier()

        def step(k, carry):
            pltpu.async_copy(rank_hbm.at[ridx_v.at[k]], rr_v.at[0], sem).wait()
            pltpu.async_copy(rank_hbm.at[cidx_v.at[k]], rc_v.at[0], sem).wait()
            ebase = wid * EPW + k * 128
            for j in range(8):
                a = rr_v[0, pl.ds(j * 16, 16)]
                b = rc_v[0, pl.ds(j * 16, 16)]
                live = (a < K) & (b < K)
                eid = ebase + j * 16 + lax.iota(jnp.int32, 16)
                r2 = jnp.where(live, a, eid & 4095)
                c2 = jnp.where(live, b, K_PAD + (eid & (P_TRASH - 1)))
                r2_v[k, pl.ds(j * 16, 16)] = r2
                c2_v[k, pl.ds(j * 16, 16)] = c2
            pltpu.sync_copy(ones_v.at[0], hist.at[c2_v.at[k]], add=True)
            return carry

        lax.fori_loop(0, IDXR, step, 0)
        pltpu.sync_copy(r2_v, r2_hbm.at[wid])
        pltpu.sync_copy(c2_v, c2_hbm.at[wid])
        plsc.subcore_barrier()
        pltpu.sync_copy(hist.at[pl.ds(s * rows_per_sub, rows_per_sub)],
                        hist_hbm.at[c, pl.ds(s * rows_per_sub, rows_per_sub)])

    return body(ridx3, cidx3, rank1, zerosk)


def _sc_scatter_rows(z, rank2):
    n_chunks = N_PAD // 128

    @functools.partial(
        pl.kernel,
        out_type=jax.ShapeDtypeStruct((K_ACC, D), jnp.float32),
        mesh=_MESH,
        scratch_types=[
            pltpu.VMEM((128, D), jnp.float32),
            pltpu.VMEM((1, 128), jnp.int32),
            pltpu.VMEM((1, 128), jnp.int32),
            pltpu.SemaphoreType.DMA,
        ],
    )
    def body(z_hbm, rank_hbm, out_hbm, rows_v, rk_v, idx_v, sem):
        c = lax.axis_index("c")
        s = lax.axis_index("s")
        wid = c * NS + s

        def step(t, carry):
            cid = wid + NW * t

            @pl.when(cid < n_chunks)
            def _():
                pltpu.sync_copy(z_hbm.at[pl.ds(cid * 128, 128)], rows_v)
                pltpu.sync_copy(rank_hbm.at[cid], rk_v.at[0])
                for j in range(8):
                    rk = rk_v[0, pl.ds(j * 16, 16)]
                    nid = cid * 128 + j * 16 + lax.iota(jnp.int32, 16)
                    idx = jnp.where(rk < K, rk, K_PAD + (nid & (P_TRASH - 1)))
                    idx_v[0, pl.ds(j * 16, 16)] = idx
                pltpu.async_copy(rows_v, out_hbm.at[idx_v.at[0]], sem).wait()

            return carry

        lax.fori_loop(0, (n_chunks + NW - 1) // NW, step, 0)

    return body(z, rank2)


_BLK = 512


def _tc_lin1(x, W, b):
    def body(x_ref, w_ref, b_ref, o_ref):
        o_ref[...] = _leaky(
            jnp.dot(x_ref[...], w_ref[...], preferred_element_type=jnp.float32)
            + b_ref[...])

    return pl.pallas_call(
        body,
        grid=(N_PAD // _BLK,),
        in_specs=[
            pl.BlockSpec((_BLK, D), lambda i: (i, 0)),
            pl.BlockSpec((D, D), lambda i: (0, 0)),
            pl.BlockSpec((1, D), lambda i: (0, 0)),
        ],
        out_specs=pl.BlockSpec((_BLK, D), lambda i: (i, 0)),
        out_shape=jax.ShapeDtypeStruct((N_PAD, D), jnp.float32),
    )(x, W, b.reshape(1, D))


def _tc_prep(h, W, d0, d1):
    n = h.shape[0]

    def body(h_ref, w_ref, d0_ref, d1_ref, y_ref, di_ref):
        dinv = lax.rsqrt(d0_ref[...] + d1_ref[...] + 1.0)
        y_ref[...] = dinv * jnp.dot(h_ref[...], w_ref[...],
                                    preferred_element_type=jnp.float32)
        di_ref[...] = dinv

    return pl.pallas_call(
        body,
        grid=(n // _BLK,),
        in_specs=[
            pl.BlockSpec((_BLK, D), lambda i: (i, 0)),
            pl.BlockSpec((D, D), lambda i: (0, 0)),
            pl.BlockSpec((_BLK, 1), lambda i: (i, 0)),
            pl.BlockSpec((_BLK, 1), lambda i: (i, 0)),
        ],
        out_specs=[
            pl.BlockSpec((_BLK, D), lambda i: (i, 0)),
            pl.BlockSpec((_BLK, 1), lambda i: (i, 0)),
        ],
        out_shape=[
            jax.ShapeDtypeStruct((n, D), jnp.float32),
            jax.ShapeDtypeStruct((n, 1), jnp.float32),
        ],
    )(h, W, d0.reshape(n, 1), d1.reshape(n, 1))


def _tc_mid(p0, p1, y_prev, dinv, b_prev, W, scale_out):
    n = y_prev.shape[0]

    def body(p0_ref, p1_ref, y_ref, di_ref, b_ref, w_ref, so_ref, o_ref):
        h = _leaky(di_ref[...] * (p0_ref[...] + p1_ref[...] + y_ref[...])
                   + b_ref[...])
        o_ref[...] = so_ref[...] * jnp.dot(h, w_ref[...],
                                           preferred_element_type=jnp.float32)

    return pl.pallas_call(
        body,
        grid=(n // _BLK,),
        in_specs=[
            pl.BlockSpec((_BLK, D), lambda i: (i, 0)),
            pl.BlockSpec((_BLK, D), lambda i: (i, 0)),
            pl.BlockSpec((_BLK, D), lambda i: (i, 0)),
            pl.BlockSpec((_BLK, 1), lambda i: (i, 0)),
            pl.BlockSpec((1, D), lambda i: (0, 0)),
            pl.BlockSpec((D, D), lambda i: (0, 0)),
            pl.BlockSpec((_BLK, 1), lambda i: (i, 0)),
        ],
        out_specs=pl.BlockSpec((_BLK, D), lambda i: (i, 0)),
        out_shape=jax.ShapeDtypeStruct((n, D), jnp.float32),
    )(p0, p1, y_prev, dinv, b_prev.reshape(1, D), W, scale_out)


def _tc_post(p0, p1, y_prev, dinv, b_prev):
    n = y_prev.shape[0]

    def body(p0_ref, p1_ref, y_ref, di_ref, b_ref, o_ref):
        o_ref[...] = _leaky(
            di_ref[...] * (p0_ref[...] + p1_ref[...] + y_ref[...]) + b_ref[...])

    return pl.pallas_call(
        body,
        grid=(n // _BLK,),
        in_specs=[
            pl.BlockSpec((_BLK, D), lambda i: (i, 0)),
            pl.BlockSpec((_BLK, D), lambda i: (i, 0)),
            pl.BlockSpec((_BLK, D), lambda i: (i, 0)),
            pl.BlockSpec((_BLK, 1), lambda i: (i, 0)),
            pl.BlockSpec((1, D), lambda i: (0, 0)),
        ],
        out_specs=pl.BlockSpec((_BLK, D), lambda i: (i, 0)),
        out_shape=jax.ShapeDtypeStruct((n, D), jnp.float32),
    )(p0, p1, y_prev, dinv, b_prev.reshape(1, D))


def _tc_score(pp0, pp1, h2, Wrel_p, Wroot_p, brel):
    def body(p0_ref, p1_ref, h_ref, wr_ref, wo_ref, b_ref, o_ref):
        aggr = p0_ref[...] + p1_ref[...]
        sc = (jnp.dot(aggr, wr_ref[...], preferred_element_type=jnp.float32)
              + jnp.dot(h_ref[...], wo_ref[...], preferred_element_type=jnp.float32)
              + b_ref[...])
        o_ref[...] = jnp.tanh(sc)

    return pl.pallas_call(
        body,
        grid=(N_PAD // _BLK,),
        in_specs=[
            pl.BlockSpec((_BLK, D), lambda i: (i, 0)),
            pl.BlockSpec((_BLK, D), lambda i: (i, 0)),
            pl.BlockSpec((_BLK, D), lambda i: (i, 0)),
            pl.BlockSpec((D, D), lambda i: (0, 0)),
            pl.BlockSpec((D, D), lambda i: (0, 0)),
            pl.BlockSpec((1, 1), lambda i: (0, 0)),
        ],
        out_specs=pl.BlockSpec((_BLK, D), lambda i: (i, 0)),
        out_shape=jax.ShapeDtypeStruct((N_PAD, D), jnp.float32),
    )(pp0, pp1, h2, Wrel_p, Wroot_p, brel.reshape(1, 1))


_JBLK = 2048


def _tc_rank(s_col, s_row):
    def body(sc_ref, sr_ref, o_ref):
        i = pl.program_id(0)
        j = pl.program_id(1)
        si = sc_ref[...]
        sj = sr_ref[...]
        ii = (lax.broadcasted_iota(jnp.int32, (_BLK, _JBLK), 0) + i * _BLK)
        jj = (lax.broadcasted_iota(jnp.int32, (_BLK, _JBLK), 1) + j * _JBLK)
        si_e = jnp.where(ii < N, si, -2.0)
        sj_e = jnp.where(jj < N, sj, -2.0)
        t = jnp.where(sj_e > si_e, 1, 0) + jnp.where(
            (sj_e == si_e) & (jj < ii), 1, 0)
        part = jnp.sum(t, axis=1, keepdims=True)

        @pl.when(j == 0)
        def _():
            o_ref[...] = jnp.zeros_like(o_ref)

        o_ref[...] += part

    return pl.pallas_call(
        body,
        grid=(N_PAD // _BLK, N_PAD // _JBLK),
        in_specs=[
            pl.BlockSpec((_BLK, 1), lambda i, j: (i, 0)),
            pl.BlockSpec((1, _JBLK), lambda i, j: (0, j)),
        ],
        out_specs=pl.BlockSpec((_BLK, 1), lambda i, j: (i, 0)),
        out_shape=jax.ShapeDtypeStruct((N_PAD, 1), jnp.int32),
    )(s_col, s_row)


def _tc_zmul(h2, s_col):
    def body(h_ref, s_ref, o_ref):
        o_ref[...] = h_ref[...] * s_ref[...]

    return pl.pallas_call(
        body,
        grid=(N_PAD // _BLK,),
        in_specs=[
            pl.BlockSpec((_BLK, D), lambda i: (i, 0)),
            pl.BlockSpec((_BLK, 1), lambda i: (i, 0)),
        ],
        out_specs=pl.BlockSpec((_BLK, D), lambda i: (i, 0)),
        out_shape=jax.ShapeDtypeStruct((N_PAD, D), jnp.float32),
    )(h2, s_col)


def _tc_final(p0, p1, y_prev, dinv, b_prev, W2, b2):
    n = y_prev.shape[0]

    def body(p0_ref, p1_ref, y_ref, di_ref, b_ref, w_ref, b2_ref, o_ref):
        h = _leaky(di_ref[...] * (p0_ref[...] + p1_ref[...] + y_ref[...])
                   + b_ref[...])
        o_ref[...] = jnp.maximum(
            jnp.dot(h, w_ref[...], preferred_element_type=jnp.float32)
            + b2_ref[...], 0.0)

    return pl.pallas_call(
        body,
        grid=(n // _BLK,),
        in_specs=[
            pl.BlockSpec((_BLK, D), lambda i: (i, 0)),
            pl.BlockSpec((_BLK, D), lambda i: (i, 0)),
            pl.BlockSpec((_BLK, D), lambda i: (i, 0)),
            pl.BlockSpec((_BLK, 1), lambda i: (i, 0)),
            pl.BlockSpec((1, D), lambda i: (0, 0)),
            pl.BlockSpec((D, D), lambda i: (0, 0)),
            pl.BlockSpec((1, D), lambda i: (0, 0)),
        ],
        out_specs=pl.BlockSpec((_BLK, D), lambda i: (i, 0)),
        out_shape=jax.ShapeDtypeStruct((n, D), jnp.float32),
    )(p0, p1, y_prev, dinv, b_prev.reshape(1, D), W2, b2.reshape(1, D))


def kernel(x, edge_index, lin1_W, lin1_b, conv1_W, conv1_b, conv2_W, conv2_b,
           pool_Wrel, pool_brel, pool_Wroot, conv3_W, conv3_b, conv4_W, conv4_b,
           conv5_W, conv5_b, conv6_W, conv6_b, lin2_W, lin2_b):
    row = edge_index[0]
    col = edge_index[1]

    epad = E_PAD - E
    pad_ids = jnp.arange(epad, dtype=jnp.int32)
    row_p = jnp.concatenate([row, pad_ids % N])
    col_p = jnp.concatenate([col, N + (pad_ids % (N_PAD - N))])
    ridx3 = row_p.reshape(NW, IDXR, 128)
    cidx3 = col_p.reshape(NW, IDXR, 128)
    rcidx4 = jnp.stack([ridx3, cidx3], axis=2)

    zeros2 = jnp.zeros((N_PAD, D), jnp.float32)
    zeros1 = jnp.zeros((N_PAD,), jnp.float32)
    zeros1k = jnp.zeros((K_ACC,), jnp.float32)

    xp = jnp.zeros((N_PAD, D), jnp.float32).at[:N].set(x)

    degp = _sc_degree(cidx3, zeros1)

    h1 = _tc_lin1(xp, lin1_W, lin1_b)

    y1, dinv1 = _tc_prep(h1, conv1_W, degp[0], degp[1])
    P1 = _sc_segsum(y1, rcidx4, zeros2, N_PAD)
    y2 = _tc_mid(P1[0], P1[1], y1, dinv1, conv1_b, conv2_W, dinv1)
    P2 = _sc_segsum(y2, rcidx4, zeros2, N_PAD)
    h2 = _tc_post(P2[0], P2[1], y2, dinv1, conv2_b)

    P3 = _sc_segsum(h2, rcidx4, zeros2, N_PAD)
    Wrel_p = jnp.zeros((D, D), jnp.float32).at[:, 0:1].set(pool_Wrel)
    Wroot_p = jnp.zeros((D, D), jnp.float32).at[:, 0:1].set(pool_Wroot)
    s_full = _tc_score(P3[0], P3[1], h2, Wrel_p, Wroot_p, pool_brel)
    s_col = s_full[:, 0:1]
    s_row = s_col.reshape(1, N_PAD)
    rank = _tc_rank(s_col, s_row)
    rank1 = rank.reshape(N_PAD)
    rank2 = rank.reshape(IDXR, 128)

    z = _tc_zmul(h2, s_col)
    r2_3, c2_3, hist2p = _sc_remap(ridx3, cidx3, rank1, zeros1k)
    rcidx4_2 = jnp.stack([r2_3, c2_3], axis=2)
    h_pool = _sc_scatter_rows(z, rank2)

    def padN(y):
        return jnp.zeros((N_PAD, D), jnp.float32).at[:K_PAD].set(y)

    hp = h_pool[:K_PAD]
    d2a = hist2p[0][:K_PAD]
    d2b = hist2p[1][:K_PAD]
    y3, dinv2 = _tc_prep(hp, conv3_W, d2a, d2b)
    P4 = _sc_segsum(padN(y3), rcidx4_2, zeros2, N_PAD)
    y4 = _tc_mid(P4[0][:K_PAD], P4[1][:K_PAD], y3, dinv2, conv3_b, conv4_W, dinv2)
    P5 = _sc_segsum(padN(y4), rcidx4_2, zeros2, N_PAD)
    y5 = _tc_mid(P5[0][:K_PAD], P5[1][:K_PAD], y4, dinv2, conv4_b, conv5_W, dinv2)
    P6 = _sc_segsum(padN(y5), rcidx4_2, zeros2, N_PAD)
    y6 = _tc_mid(P6[0][:K_PAD], P6[1][:K_PAD], y5, dinv2, conv5_b, conv6_W, dinv2)
    P7 = _sc_segsum(padN(y6), rcidx4_2, zeros2, N_PAD)
    out = _tc_final(P7[0][:K_PAD], P7[1][:K_PAD], y6, dinv2, conv6_b,
                    lin2_W, lin2_b)
    return out[:K]


# speedup vs baseline: 21.9244x; 1.2148x over previous
import functools

import jax
import jax.numpy as jnp
from jax import lax
from jax.experimental import pallas as pl
from jax.experimental.pallas import tpu as pltpu
from jax.experimental.pallas import tpu_sc as plsc

N = 10000
E = 320000
D = 128
K = 5000

NC = 2
NS = 16
NW = NC * NS

N_PAD = 10240
EPW = 10240
E_PAD = NW * EPW
IDXR = EPW // 128

K_PAD = 5120
P_TRASH = 1024
K_ACC = K_PAD + P_TRASH

_MESH = plsc.VectorSubcoreMesh(core_axis_name="c", subcore_axis_name="s")


def _leaky(v):
    return jnp.where(v > 0, v, 0.01 * v)


def _sc_degree(cidx3, zeros1):
    @functools.partial(
        pl.kernel,
        out_type=jax.ShapeDtypeStruct((NC, N_PAD), jnp.float32),
        mesh=_MESH,
        scratch_types=[
            pltpu.VMEM((IDXR, 128), jnp.int32),
            pltpu.VMEM((1, 128), jnp.float32),
            pltpu.VMEM_SHARED((N_PAD,), jnp.float32),
            pltpu.SemaphoreType.DMA,
        ],
    )
    def body(cidx_hbm, zeros_hbm, out_hbm, cidx_v, ones_v, hist, sem):
        c = lax.axis_index("c")
        s = lax.axis_index("s")
        wid = c * NS + s
        rows_per_sub = N_PAD // NS
        pltpu.sync_copy(zeros_hbm.at[pl.ds(s * rows_per_sub, rows_per_sub)],
                        hist.at[pl.ds(s * rows_per_sub, rows_per_sub)])
        pltpu.sync_copy(cidx_hbm.at[wid], cidx_v)
        for j in range(8):
            ones_v[0, pl.ds(j * 16, 16)] = jnp.ones((16,), jnp.float32)
        plsc.subcore_barrier()

        def step(k, carry):
            pltpu.sync_copy(ones_v.at[0], hist.at[cidx_v.at[k]], add=True)
            return carry

        lax.fori_loop(0, IDXR, step, 0)
        plsc.subcore_barrier()
        pltpu.sync_copy(hist.at[pl.ds(s * rows_per_sub, rows_per_sub)],
                        out_hbm.at[c, pl.ds(s * rows_per_sub, rows_per_sub)])

    return body(cidx3, zeros1)


def _sc_segsum(y, rcidx4, counts, zeros2, n_acc):
    @functools.partial(
        pl.kernel,
        out_type=jax.ShapeDtypeStruct((NC, n_acc, D), jnp.float32),
        mesh=_MESH,
        scratch_types=[
            [pltpu.VMEM((2, 128), jnp.int32) for _ in range(4)],
            pltpu.VMEM((128, D), jnp.float32),
            pltpu.VMEM((128, D), jnp.float32),
            pltpu.VMEM((1, 16), jnp.int32),
            pltpu.VMEM_SHARED((n_acc, D), jnp.float32),
            [pltpu.SemaphoreType.DMA for _ in range(4)],
            [pltpu.SemaphoreType.DMA for _ in range(2)],
            [pltpu.SemaphoreType.DMA for _ in range(2)],
        ],
    )
    def body(y_hbm, rc_hbm, cnt_hbm, zeros_hbm, out_hbm,
             idx, buf0, buf1, cnt_v, acc, isem, gsem, ssem):
        c = lax.axis_index("c")
        s = lax.axis_index("s")
        wid = c * NS + s
        buf = (buf0, buf1)
        rows_per_sub = n_acc // NS
        pltpu.sync_copy(cnt_hbm.at[wid], cnt_v.at[0])
        pltpu.sync_copy(zeros_hbm.at[pl.ds(s * rows_per_sub, rows_per_sub)],
                        acc.at[pl.ds(s * rows_per_sub, rows_per_sub)])
        nch = cnt_v[0, pl.ds(0, 16)][0]
        plsc.subcore_barrier()

        pltpu.async_copy(rc_hbm.at[wid, 0], idx[0], isem[0])
        pltpu.async_copy(rc_hbm.at[wid, 1], idx[1], isem[1])

        def phase(k, K, kk):
            b = K % 2
            pb = (K - 1) % 2
            pq = (K - 1) % 4
            nq = (K + 2) % 4
            pltpu.make_async_copy(rc_hbm.at[wid, k], idx[K], isem[K]).wait()

            @pl.when(k >= 2)
            def _():
                pltpu.make_async_copy(buf[b], acc.at[idx[nq].at[1]], ssem[b]).wait()

            pltpu.async_copy(y_hbm.at[idx[K].at[0]], buf[b], gsem[b])

            @pl.when(k >= 1)
            def _():
                pltpu.make_async_copy(y_hbm.at[idx[pq].at[0]], buf[pb],
                                      gsem[pb]).wait()
                pltpu.async_copy(buf[pb], acc.at[idx[pq].at[1]], ssem[pb],
                                 add=True)

            @pl.when(k + 2 < nch)
            def _():
                pltpu.async_copy(rc_hbm.at[wid, k + 2], idx[nq], isem[nq])

        def step(kk, carry):
            for K in range(4):
                phase(4 * kk + K, K, kk)
            return carry

        lax.fori_loop(0, nch // 4, step, 0)
        pltpu.make_async_copy(y_hbm.at[idx[3].at[0]], buf1, gsem[1]).wait()
        pltpu.sync_copy(buf1, acc.at[idx[3].at[1]], add=True)
        pltpu.make_async_copy(buf0, acc.at[idx[2].at[1]], ssem[0]).wait()
        plsc.subcore_barrier()
        pltpu.sync_copy(acc.at[pl.ds(s * rows_per_sub, rows_per_sub)],
                        out_hbm.at[c, pl.ds(s * rows_per_sub, rows_per_sub)])

    return body(y, rcidx4, counts, zeros2)


_FLATW = IDXR * 2 * 128
_PARK = _FLATW


def _sc_remap(ridx3, cidx3, rank1, zerosk):
    @functools.partial(
        pl.kernel,
        out_type=(
            jax.ShapeDtypeStruct((NW, _FLATW), jnp.int32),
            jax.ShapeDtypeStruct((NW, 16), jnp.int32),
            jax.ShapeDtypeStruct((NC, K_ACC), jnp.float32),
        ),
        mesh=_MESH,
        scratch_types=[
            pltpu.VMEM((IDXR, 128), jnp.int32),
            pltpu.VMEM((IDXR, 128), jnp.int32),
            pltpu.VMEM((1, 128), jnp.int32),
            pltpu.VMEM((1, 128), jnp.int32),
            pltpu.VMEM((1, 128), jnp.int32),
            pltpu.VMEM((1, 128), jnp.int32),
            pltpu.VMEM((1, 128), jnp.int32),
            pltpu.VMEM((1, 128), jnp.int32),
            pltpu.VMEM((1, 128), jnp.int32),
            pltpu.VMEM((256,), jnp.int32),
            pltpu.VMEM((1, 128), jnp.float32),
            pltpu.VMEM((1, 16), jnp.int32),
            pltpu.VMEM((1, 48), jnp.int32),
            pltpu.VMEM_SHARED((NS * (_FLATW + 256),), jnp.int32),
            pltpu.VMEM_SHARED((K_ACC,), jnp.float32),
            pltpu.SemaphoreType.DMA,
        ],
    )
    def body(ridx_hbm, cidx_hbm, rank_hbm, zeros_hbm, rc2_hbm, cnt_hbm,
             hist_hbm, ridx_v, cidx_v, rr_v, rc_v, av_s, bv_s, fr_s, fc_s,
             ch_s, fb_s, ones_v, cnt_v, scr, flat_sh, hist, sem):
        c = lax.axis_index("c")
        s = lax.axis_index("s")
        wid = c * NS + s
        rows_per_sub = K_ACC // NS
        pltpu.sync_copy(zeros_hbm.at[pl.ds(s * rows_per_sub, rows_per_sub)],
                        hist.at[pl.ds(s * rows_per_sub, rows_per_sub)])
        pltpu.sync_copy(ridx_hbm.at[wid], ridx_v)
        pltpu.sync_copy(cidx_hbm.at[wid], cidx_v)
        for j in range(8):
            ones_v[0, pl.ds(j * 16, 16)] = jnp.ones((16,), jnp.float32)
        scr[0, pl.ds(0, 16)] = jnp.zeros((16,), jnp.int32)
        plsc.subcore_barrier()

        def step(k, off):
            pltpu.async_copy(rank_hbm.at[ridx_v.at[k]], rr_v.at[0], sem).wait()
            pltpu.async_copy(rank_hbm.at[cidx_v.at[k]], rc_v.at[0], sem).wait()
            lane = lax.iota(jnp.int32, 16)
            for j in range(8):
                a = rr_v[0, pl.ds(j * 16, 16)]
                b = rc_v[0, pl.ds(j * 16, 16)]
                live = (a < K) & (b < K)
                liveint = jnp.where(live, 1, 0)
                cum = liveint
                for d in (1, 2, 4, 8):
                    scr[0, pl.ds(16, 16)] = cum
                    cum = cum + scr[0, pl.ds(16 - d, 16)]
                scr[0, pl.ds(16, 16)] = cum
                cnt_g = scr[0, pl.ds(31, 16)][0]
                pos = off + cum - liveint
                base = s * (_FLATW + 256)
                fr = base + ((pos >> 7) << 8) + (pos & 127)
                park = base + _PARK
                av_s[0, pl.ds(j * 16, 16)] = a
                bv_s[0, pl.ds(j * 16, 16)] = b
                fr_s[0, pl.ds(j * 16, 16)] = jnp.where(live, fr, park + lane)
                fc_s[0, pl.ds(j * 16, 16)] = jnp.where(live, fr + 128,
                                                       park + 128 + lane)
                ch_s[0, pl.ds(j * 16, 16)] = jnp.where(
                    live, b, K_PAD + ((k * 128 + j * 16 + lane) & (P_TRASH - 1)))
                off = off + cnt_g
            pltpu.sync_copy(av_s.at[0], flat_sh.at[fr_s.at[0]])
            pltpu.sync_copy(bv_s.at[0], flat_sh.at[fc_s.at[0]])
            pltpu.sync_copy(ones_v.at[0], hist.at[ch_s.at[0]], add=True)
            return off

        cnt = lax.fori_loop(0, IDXR, step, jnp.int32(0))

        nch = jnp.maximum(((cnt + 511) // 512) * 4, 4)

        def fill(cidx, carry):
            pltpu.sync_copy(flat_sh.at[pl.ds(s * (_FLATW + 256) + cidx * 256, 256)], fb_s)
            lane = lax.iota(jnp.int32, 16)
            for j in range(8):
                slot = cidx * 128 + j * 16 + lane
                keep = slot < cnt
                cur_r = fb_s[pl.ds(j * 16, 16)]
                cur_c = fb_s[pl.ds(128 + j * 16, 16)]
                fb_s[pl.ds(j * 16, 16)] = jnp.where(keep, cur_r, slot & 4095)
                fb_s[pl.ds(128 + j * 16, 16)] = jnp.where(
                    keep, cur_c, K_PAD + (slot & (P_TRASH - 1)))
            pltpu.sync_copy(fb_s, flat_sh.at[pl.ds(s * (_FLATW + 256) + cidx * 256, 256)])
            return carry

        lax.fori_loop(cnt >> 7, nch, fill, 0)

        cnt_v[0, pl.ds(0, 16)] = jnp.broadcast_to(nch, (16,))
        pltpu.sync_copy(cnt_v.at[0], cnt_hbm.at[wid])
        pltpu.sync_copy(flat_sh.at[pl.ds(s * (_FLATW + 256), _FLATW)], rc2_hbm.at[wid])
        plsc.subcore_barrier()
        pltpu.sync_copy(hist.at[pl.ds(s * rows_per_sub, rows_per_sub)],
                        hist_hbm.at[c, pl.ds(s * rows_per_sub, rows_per_sub)])

    return body(ridx3, cidx3, rank1, zerosk)


def _sc_scatter_rows(z, rank2):
    n_chunks = N_PAD // 128

    @functools.partial(
        pl.kernel,
        out_type=jax.ShapeDtypeStruct((K_ACC, D), jnp.float32),
        mesh=_MESH,
        scratch_types=[
            pltpu.VMEM((128, D), jnp.float32),
            pltpu.VMEM((1, 128), jnp.int32),
            pltpu.VMEM((1, 128), jnp.int32),
            pltpu.SemaphoreType.DMA,
        ],
    )
    def body(z_hbm, rank_hbm, out_hbm, rows_v, rk_v, idx_v, sem):
        c = lax.axis_index("c")
        s = lax.axis_index("s")
        wid = c * NS + s

        def step(t, carry):
            cid = wid + NW * t

            @pl.when(cid < n_chunks)
            def _():
                pltpu.sync_copy(z_hbm.at[pl.ds(cid * 128, 128)], rows_v)
                pltpu.sync_copy(rank_hbm.at[cid], rk_v.at[0])
                for j in range(8):
                    rk = rk_v[0, pl.ds(j * 16, 16)]
                    nid = cid * 128 + j * 16 + lax.iota(jnp.int32, 16)
                    idx = jnp.where(rk < K, rk, K_PAD + (nid & (P_TRASH - 1)))
                    idx_v[0, pl.ds(j * 16, 16)] = idx
                pltpu.async_copy(rows_v, out_hbm.at[idx_v.at[0]], sem).wait()

            return carry

        lax.fori_loop(0, (n_chunks + NW - 1) // NW, step, 0)

    return body(z, rank2)


_BLK = 512


def _tc_lin1(x, W, b):
    def body(x_ref, w_ref, b_ref, o_ref):
        o_ref[...] = _leaky(
            jnp.dot(x_ref[...], w_ref[...], preferred_element_type=jnp.float32)
            + b_ref[...])

    return pl.pallas_call(
        body,
        grid=(N_PAD // _BLK,),
        in_specs=[
            pl.BlockSpec((_BLK, D), lambda i: (i, 0)),
            pl.BlockSpec((D, D), lambda i: (0, 0)),
            pl.BlockSpec((1, D), lambda i: (0, 0)),
        ],
        out_specs=pl.BlockSpec((_BLK, D), lambda i: (i, 0)),
        out_shape=jax.ShapeDtypeStruct((N_PAD, D), jnp.float32),
    )(x, W, b.reshape(1, D))


def _tc_prep(h, W, d0, d1):
    n = h.shape[0]

    def body(h_ref, w_ref, d0_ref, d1_ref, y_ref, di_ref):
        dinv = lax.rsqrt(d0_ref[...] + d1_ref[...] + 1.0)
        y_ref[...] = dinv * jnp.dot(h_ref[...], w_ref[...],
                                    preferred_element_type=jnp.float32)
        di_ref[...] = dinv

    return pl.pallas_call(
        body,
        grid=(n // _BLK,),
        in_specs=[
            pl.BlockSpec((_BLK, D), lambda i: (i, 0)),
            pl.BlockSpec((D, D), lambda i: (0, 0)),
            pl.BlockSpec((_BLK, 1), lambda i: (i, 0)),
            pl.BlockSpec((_BLK, 1), lambda i: (i, 0)),
        ],
        out_specs=[
            pl.BlockSpec((_BLK, D), lambda i: (i, 0)),
            pl.BlockSpec((_BLK, 1), lambda i: (i, 0)),
        ],
        out_shape=[
            jax.ShapeDtypeStruct((n, D), jnp.float32),
            jax.ShapeDtypeStruct((n, 1), jnp.float32),
        ],
    )(h, W, d0.reshape(n, 1), d1.reshape(n, 1))


def _tc_mid(p0, p1, y_prev, dinv, b_prev, W, scale_out):
    n = y_prev.shape[0]

    def body(p0_ref, p1_ref, y_ref, di_ref, b_ref, w_ref, so_ref, o_ref):
        h = _leaky(di_ref[...] * (p0_ref[...] + p1_ref[...] + y_ref[...])
                   + b_ref[...])
        o_ref[...] = so_ref[...] * jnp.dot(h, w_ref[...],
                                           preferred_element_type=jnp.float32)

    return pl.pallas_call(
        body,
        grid=(n // _BLK,),
        in_specs=[
            pl.BlockSpec((_BLK, D), lambda i: (i, 0)),
            pl.BlockSpec((_BLK, D), lambda i: (i, 0)),
            pl.BlockSpec((_BLK, D), lambda i: (i, 0)),
            pl.BlockSpec((_BLK, 1), lambda i: (i, 0)),
            pl.BlockSpec((1, D), lambda i: (0, 0)),
            pl.BlockSpec((D, D), lambda i: (0, 0)),
            pl.BlockSpec((_BLK, 1), lambda i: (i, 0)),
        ],
        out_specs=pl.BlockSpec((_BLK, D), lambda i: (i, 0)),
        out_shape=jax.ShapeDtypeStruct((n, D), jnp.float32),
    )(p0, p1, y_prev, dinv, b_prev.reshape(1, D), W, scale_out)


def _tc_post(p0, p1, y_prev, dinv, b_prev):
    n = y_prev.shape[0]

    def body(p0_ref, p1_ref, y_ref, di_ref, b_ref, o_ref):
        o_ref[...] = _leaky(
            di_ref[...] * (p0_ref[...] + p1_ref[...] + y_ref[...]) + b_ref[...])

    return pl.pallas_call(
        body,
        grid=(n // _BLK,),
        in_specs=[
            pl.BlockSpec((_BLK, D), lambda i: (i, 0)),
            pl.BlockSpec((_BLK, D), lambda i: (i, 0)),
            pl.BlockSpec((_BLK, D), lambda i: (i, 0)),
            pl.BlockSpec((_BLK, 1), lambda i: (i, 0)),
            pl.BlockSpec((1, D), lambda i: (0, 0)),
        ],
        out_specs=pl.BlockSpec((_BLK, D), lambda i: (i, 0)),
        out_shape=jax.ShapeDtypeStruct((n, D), jnp.float32),
    )(p0, p1, y_prev, dinv, b_prev.reshape(1, D))


def _tc_score(pp0, pp1, h2, Wrel_p, Wroot_p, brel):
    def body(p0_ref, p1_ref, h_ref, wr_ref, wo_ref, b_ref, o_ref):
        aggr = p0_ref[...] + p1_ref[...]
        sc = (jnp.dot(aggr, wr_ref[...], preferred_element_type=jnp.float32)
              + jnp.dot(h_ref[...], wo_ref[...], preferred_element_type=jnp.float32)
              + b_ref[...])
        o_ref[...] = jnp.tanh(sc)

    return pl.pallas_call(
        body,
        grid=(N_PAD // _BLK,),
        in_specs=[
            pl.BlockSpec((_BLK, D), lambda i: (i, 0)),
            pl.BlockSpec((_BLK, D), lambda i: (i, 0)),
            pl.BlockSpec((_BLK, D), lambda i: (i, 0)),
            pl.BlockSpec((D, D), lambda i: (0, 0)),
            pl.BlockSpec((D, D), lambda i: (0, 0)),
            pl.BlockSpec((1, 1), lambda i: (0, 0)),
        ],
        out_specs=pl.BlockSpec((_BLK, D), lambda i: (i, 0)),
        out_shape=jax.ShapeDtypeStruct((N_PAD, D), jnp.float32),
    )(pp0, pp1, h2, Wrel_p, Wroot_p, brel.reshape(1, 1))


_JBLK = 2048


def _tc_rank(s_col, s_row):
    def body(sc_ref, sr_ref, o_ref):
        i = pl.program_id(0)
        j = pl.program_id(1)
        si = sc_ref[...]
        sj = sr_ref[...]
        ii = (lax.broadcasted_iota(jnp.int32, (_BLK, _JBLK), 0) + i * _BLK)
        jj = (lax.broadcasted_iota(jnp.int32, (_BLK, _JBLK), 1) + j * _JBLK)
        si_e = jnp.where(ii < N, si, -2.0)
        sj_e = jnp.where(jj < N, sj, -2.0)
        t = jnp.where(sj_e > si_e, 1, 0) + jnp.where(
            (sj_e == si_e) & (jj < ii), 1, 0)
        part = jnp.sum(t, axis=1, keepdims=True)

        @pl.when(j == 0)
        def _():
            o_ref[...] = jnp.zeros_like(o_ref)

        o_ref[...] += part

    return pl.pallas_call(
        body,
        grid=(N_PAD // _BLK, N_PAD // _JBLK),
        in_specs=[
            pl.BlockSpec((_BLK, 1), lambda i, j: (i, 0)),
            pl.BlockSpec((1, _JBLK), lambda i, j: (0, j)),
        ],
        out_specs=pl.BlockSpec((_BLK, 1), lambda i, j: (i, 0)),
        out_shape=jax.ShapeDtypeStruct((N_PAD, 1), jnp.int32),
    )(s_col, s_row)


def _tc_zmul(h2, s_col):
    def body(h_ref, s_ref, o_ref):
        o_ref[...] = h_ref[...] * s_ref[...]

    return pl.pallas_call(
        body,
        grid=(N_PAD // _BLK,),
        in_specs=[
            pl.BlockSpec((_BLK, D), lambda i: (i, 0)),
            pl.BlockSpec((_BLK, 1), lambda i: (i, 0)),
        ],
        out_specs=pl.BlockSpec((_BLK, D), lambda i: (i, 0)),
        out_shape=jax.ShapeDtypeStruct((N_PAD, D), jnp.float32),
    )(h2, s_col)


def _tc_final(p0, p1, y_prev, dinv, b_prev, W2, b2):
    n = y_prev.shape[0]

    def body(p0_ref, p1_ref, y_ref, di_ref, b_ref, w_ref, b2_ref, o_ref):
        h = _leaky(di_ref[...] * (p0_ref[...] + p1_ref[...] + y_ref[...])
                   + b_ref[...])
        o_ref[...] = jnp.maximum(
            jnp.dot(h, w_ref[...], preferred_element_type=jnp.float32)
            + b2_ref[...], 0.0)

    return pl.pallas_call(
        body,
        grid=(n // _BLK,),
        in_specs=[
            pl.BlockSpec((_BLK, D), lambda i: (i, 0)),
            pl.BlockSpec((_BLK, D), lambda i: (i, 0)),
            pl.BlockSpec((_BLK, D), lambda i: (i, 0)),
            pl.BlockSpec((_BLK, 1), lambda i: (i, 0)),
            pl.BlockSpec((1, D), lambda i: (0, 0)),
            pl.BlockSpec((D, D), lambda i: (0, 0)),
            pl.BlockSpec((1, D), lambda i: (0, 0)),
        ],
        out_specs=pl.BlockSpec((_BLK, D), lambda i: (i, 0)),
        out_shape=jax.ShapeDtypeStruct((n, D), jnp.float32),
    )(p0, p1, y_prev, dinv, b_prev.reshape(1, D), W2, b2.reshape(1, D))


def kernel(x, edge_index, lin1_W, lin1_b, conv1_W, conv1_b, conv2_W, conv2_b,
           pool_Wrel, pool_brel, pool_Wroot, conv3_W, conv3_b, conv4_W, conv4_b,
           conv5_W, conv5_b, conv6_W, conv6_b, lin2_W, lin2_b):
    row = edge_index[0]
    col = edge_index[1]

    epad = E_PAD - E
    pad_ids = jnp.arange(epad, dtype=jnp.int32)
    row_p = jnp.concatenate([row, pad_ids % N])
    col_p = jnp.concatenate([col, N + (pad_ids % (N_PAD - N))])
    ridx3 = row_p.reshape(NW, IDXR, 128)
    cidx3 = col_p.reshape(NW, IDXR, 128)
    rcidx4 = jnp.stack([ridx3, cidx3], axis=2)
    counts_full = jnp.full((NW, 16), IDXR, jnp.int32)

    zeros2 = jnp.zeros((N_PAD, D), jnp.float32)
    zeros1 = jnp.zeros((N_PAD,), jnp.float32)
    zeros1k = jnp.zeros((K_ACC,), jnp.float32)

    xp = jnp.zeros((N_PAD, D), jnp.float32).at[:N].set(x)

    degp = _sc_degree(cidx3, zeros1)

    h1 = _tc_lin1(xp, lin1_W, lin1_b)

    y1, dinv1 = _tc_prep(h1, conv1_W, degp[0], degp[1])
    P1 = _sc_segsum(y1, rcidx4, counts_full, zeros2, N_PAD)
    y2 = _tc_mid(P1[0], P1[1], y1, dinv1, conv1_b, conv2_W, dinv1)
    P2 = _sc_segsum(y2, rcidx4, counts_full, zeros2, N_PAD)
    h2 = _tc_post(P2[0], P2[1], y2, dinv1, conv2_b)

    P3 = _sc_segsum(h2, rcidx4, counts_full, zeros2, N_PAD)
    Wrel_p = jnp.zeros((D, D), jnp.float32).at[:, 0:1].set(pool_Wrel)
    Wroot_p = jnp.zeros((D, D), jnp.float32).at[:, 0:1].set(pool_Wroot)
    s_full = _tc_score(P3[0], P3[1], h2, Wrel_p, Wroot_p, pool_brel)
    s_col = s_full[:, 0:1]
    s_row = s_col.reshape(1, N_PAD)
    rank = _tc_rank(s_col, s_row)
    rank1 = rank.reshape(N_PAD)
    rank2 = rank.reshape(IDXR, 128)

    z = _tc_zmul(h2, s_col)
    rc2_flat, counts2, hist2p = _sc_remap(ridx3, cidx3, rank1, zeros1k)
    rcidx4_2 = rc2_flat.reshape(NW, IDXR, 2, 128)
    h_pool = _sc_scatter_rows(z, rank2)

    def padN(y):
        return jnp.zeros((N_PAD, D), jnp.float32).at[:K_PAD].set(y)

    hp = h_pool[:K_PAD]
    d2a = hist2p[0][:K_PAD]
    d2b = hist2p[1][:K_PAD]
    y3, dinv2 = _tc_prep(hp, conv3_W, d2a, d2b)
    P4 = _sc_segsum(padN(y3), rcidx4_2, counts2, zeros2, N_PAD)
    y4 = _tc_mid(P4[0][:K_PAD], P4[1][:K_PAD], y3, dinv2, conv3_b, conv4_W, dinv2)
    P5 = _sc_segsum(padN(y4), rcidx4_2, counts2, zeros2, N_PAD)
    y5 = _tc_mid(P5[0][:K_PAD], P5[1][:K_PAD], y4, dinv2, conv4_b, conv5_W, dinv2)
    P6 = _sc_segsum(padN(y5), rcidx4_2, counts2, zeros2, N_PAD)
    y6 = _tc_mid(P6[0][:K_PAD], P6[1][:K_PAD], y5, dinv2, conv5_b, conv6_W, dinv2)
    P7 = _sc_segsum(padN(y6), rcidx4_2, counts2, zeros2, N_PAD)
    out = _tc_final(P7[0][:K_PAD], P7[1][:K_PAD], y6, dinv2, conv6_b,
                    lin2_W, lin2_b)
    return out[:K]


# speedup vs baseline: 24.1127x; 1.0998x over previous
import functools

import jax
import jax.numpy as jnp
from jax import lax
from jax.experimental import pallas as pl
from jax.experimental.pallas import tpu as pltpu
from jax.experimental.pallas import tpu_sc as plsc

N = 10000
E = 320000
D = 128
K = 5000

NC = 2
NS = 16
NW = NC * NS

N_PAD = 10240
EPW = 10240
E_PAD = NW * EPW
IDXR = EPW // 128

K_PAD = 5120
P_TRASH = 1024
K_ACC = K_PAD + P_TRASH

_MESH = plsc.VectorSubcoreMesh(core_axis_name="c", subcore_axis_name="s")


def _leaky(v):
    return jnp.where(v > 0, v, 0.01 * v)


def _sc_degree(cidx3, zeros1):
    @functools.partial(
        pl.kernel,
        out_type=jax.ShapeDtypeStruct((NC, N_PAD), jnp.float32),
        mesh=_MESH,
        scratch_types=[
            pltpu.VMEM((IDXR, 128), jnp.int32),
            pltpu.VMEM((1, 128), jnp.float32),
            pltpu.VMEM_SHARED((N_PAD,), jnp.float32),
            pltpu.SemaphoreType.DMA,
        ],
    )
    def body(cidx_hbm, zeros_hbm, out_hbm, cidx_v, ones_v, hist, sem):
        c = lax.axis_index("c")
        s = lax.axis_index("s")
        wid = c * NS + s
        rows_per_sub = N_PAD // NS
        pltpu.sync_copy(zeros_hbm.at[pl.ds(s * rows_per_sub, rows_per_sub)],
                        hist.at[pl.ds(s * rows_per_sub, rows_per_sub)])
        pltpu.sync_copy(cidx_hbm.at[wid], cidx_v)
        for j in range(8):
            ones_v[0, pl.ds(j * 16, 16)] = jnp.ones((16,), jnp.float32)
        plsc.subcore_barrier()

        def step(k, carry):
            pltpu.sync_copy(ones_v.at[0], hist.at[cidx_v.at[k]], add=True)
            return carry

        lax.fori_loop(0, IDXR, step, 0)
        plsc.subcore_barrier()
        pltpu.sync_copy(hist.at[pl.ds(s * rows_per_sub, rows_per_sub)],
                        out_hbm.at[c, pl.ds(s * rows_per_sub, rows_per_sub)])

    return body(cidx3, zeros1)


def _sc_segsum(y, rcidx4, counts, zeros2, n_acc):
    @functools.partial(
        pl.kernel,
        out_type=jax.ShapeDtypeStruct((NC, n_acc, D), jnp.float32),
        mesh=_MESH,
        scratch_types=[
            [pltpu.VMEM((2, 128), jnp.int32) for _ in range(4)],
            pltpu.VMEM((128, D), jnp.float32),
            pltpu.VMEM((128, D), jnp.float32),
            pltpu.VMEM((1, 16), jnp.int32),
            pltpu.VMEM_SHARED((n_acc, D), jnp.float32),
            [pltpu.SemaphoreType.DMA for _ in range(4)],
            [pltpu.SemaphoreType.DMA for _ in range(2)],
            [pltpu.SemaphoreType.DMA for _ in range(2)],
        ],
    )
    def body(y_hbm, rc_hbm, cnt_hbm, zeros_hbm, out_hbm,
             idx, buf0, buf1, cnt_v, acc, isem, gsem, ssem):
        c = lax.axis_index("c")
        s = lax.axis_index("s")
        wid = c * NS + s
        buf = (buf0, buf1)
        rows_per_sub = n_acc // NS
        pltpu.sync_copy(cnt_hbm.at[wid], cnt_v.at[0])
        pltpu.sync_copy(zeros_hbm.at[pl.ds(s * rows_per_sub, rows_per_sub)],
                        acc.at[pl.ds(s * rows_per_sub, rows_per_sub)])
        nch = cnt_v[0, pl.ds(0, 16)][0]
        plsc.subcore_barrier()

        pltpu.async_copy(rc_hbm.at[wid, 0], idx[0], isem[0])
        pltpu.async_copy(rc_hbm.at[wid, 1], idx[1], isem[1])

        def phase(k, K, kk):
            b = K % 2
            pb = (K - 1) % 2
            pq = (K - 1) % 4
            nq = (K + 2) % 4
            pltpu.make_async_copy(rc_hbm.at[wid, k], idx[K], isem[K]).wait()

            @pl.when(k >= 2)
            def _():
                pltpu.make_async_copy(buf[b], acc.at[idx[nq].at[1]], ssem[b]).wait()

            pltpu.async_copy(y_hbm.at[idx[K].at[0]], buf[b], gsem[b])

            @pl.when(k >= 1)
            def _():
                pltpu.make_async_copy(y_hbm.at[idx[pq].at[0]], buf[pb],
                                      gsem[pb]).wait()
                pltpu.async_copy(buf[pb], acc.at[idx[pq].at[1]], ssem[pb],
                                 add=True)

            @pl.when(k + 2 < nch)
            def _():
                pltpu.async_copy(rc_hbm.at[wid, k + 2], idx[nq], isem[nq])

        def step(kk, carry):
            for K in range(4):
                phase(4 * kk + K, K, kk)
            return carry

        lax.fori_loop(0, nch // 4, step, 0)
        pltpu.make_async_copy(y_hbm.at[idx[3].at[0]], buf1, gsem[1]).wait()
        pltpu.sync_copy(buf1, acc.at[idx[3].at[1]], add=True)
        pltpu.make_async_copy(buf0, acc.at[idx[2].at[1]], ssem[0]).wait()
        plsc.subcore_barrier()
        pltpu.sync_copy(acc.at[pl.ds(s * rows_per_sub, rows_per_sub)],
                        out_hbm.at[c, pl.ds(s * rows_per_sub, rows_per_sub)])

    return body(y, rcidx4, counts, zeros2)


_FLATW = IDXR * 2 * 128
_PARK = _FLATW


def _sc_remap(ridx3, cidx3, rank1, zerosk):
    @functools.partial(
        pl.kernel,
        out_type=(
            jax.ShapeDtypeStruct((NW, _FLATW), jnp.int32),
            jax.ShapeDtypeStruct((NW, 16), jnp.int32),
            jax.ShapeDtypeStruct((NC, K_ACC), jnp.float32),
        ),
        mesh=_MESH,
        scratch_types=[
            pltpu.VMEM((IDXR, 128), jnp.int32),
            pltpu.VMEM((IDXR, 128), jnp.int32),
            pltpu.VMEM((2, 128), jnp.int32),
            pltpu.VMEM((2, 128), jnp.int32),
            pltpu.VMEM((2, 128), jnp.int32),
            pltpu.VMEM((2, 128), jnp.int32),
            pltpu.VMEM((2, 128), jnp.int32),
            pltpu.VMEM((2, 128), jnp.int32),
            pltpu.VMEM((2, 128), jnp.int32),
            pltpu.VMEM((256,), jnp.int32),
            pltpu.VMEM((1, 128), jnp.float32),
            pltpu.VMEM((1, 16), jnp.int32),
            pltpu.VMEM((1, 48), jnp.int32),
            pltpu.VMEM_SHARED((NS * (_FLATW + 256),), jnp.int32),
            pltpu.VMEM_SHARED((K_ACC,), jnp.float32),
            [pltpu.SemaphoreType.DMA for _ in range(2)],
            [pltpu.SemaphoreType.DMA for _ in range(2)],
        ],
    )
    def body(ridx_hbm, cidx_hbm, rank_hbm, zeros_hbm, rc2_hbm, cnt_hbm,
             hist_hbm, ridx_v, cidx_v, rr_v, rc_v, av_s, bv_s, fr_s, fc_s,
             ch_s, fb_s, ones_v, cnt_v, scr, flat_sh, hist, gsem, ssem):
        c = lax.axis_index("c")
        s = lax.axis_index("s")
        wid = c * NS + s
        rows_per_sub = K_ACC // NS
        pltpu.sync_copy(zeros_hbm.at[pl.ds(s * rows_per_sub, rows_per_sub)],
                        hist.at[pl.ds(s * rows_per_sub, rows_per_sub)])
        pltpu.sync_copy(ridx_hbm.at[wid], ridx_v)
        pltpu.sync_copy(cidx_hbm.at[wid], cidx_v)
        for j in range(8):
            ones_v[0, pl.ds(j * 16, 16)] = jnp.ones((16,), jnp.float32)
        scr[0, pl.ds(0, 16)] = jnp.zeros((16,), jnp.int32)
        plsc.subcore_barrier()

        pltpu.async_copy(rank_hbm.at[ridx_v.at[0]], rr_v.at[0], gsem[0])
        pltpu.async_copy(rank_hbm.at[cidx_v.at[0]], rc_v.at[0], gsem[0])
        pltpu.async_copy(rank_hbm.at[ridx_v.at[1]], rr_v.at[1], gsem[1])
        pltpu.async_copy(rank_hbm.at[cidx_v.at[1]], rc_v.at[1], gsem[1])

        def chunk(k, p, off):
            pltpu.make_async_copy(rank_hbm.at[ridx_v.at[k]], rr_v.at[p],
                                  gsem[p]).wait()
            pltpu.make_async_copy(rank_hbm.at[cidx_v.at[k]], rc_v.at[p],
                                  gsem[p]).wait()

            @pl.when(k >= 2)
            def _():
                pltpu.make_async_copy(av_s.at[p], flat_sh.at[fr_s.at[p]],
                                      ssem[p]).wait()
                pltpu.make_async_copy(bv_s.at[p], flat_sh.at[fc_s.at[p]],
                                      ssem[p]).wait()
                pltpu.make_async_copy(ones_v.at[0], hist.at[ch_s.at[p]],
                                      ssem[p]).wait()

            lane = lax.iota(jnp.int32, 16)
            for j in range(8):
                a = rr_v[p, pl.ds(j * 16, 16)]
                b = rc_v[p, pl.ds(j * 16, 16)]
                live = (a < K) & (b < K)
                liveint = jnp.where(live, 1, 0)
                cum = liveint
                for d in (1, 2, 4, 8):
                    scr[0, pl.ds(16, 16)] = cum
                    cum = cum + scr[0, pl.ds(16 - d, 16)]
                scr[0, pl.ds(16, 16)] = cum
                cnt_g = scr[0, pl.ds(31, 16)][0]
                pos = off + cum - liveint
                base = s * (_FLATW + 256)
                fr = base + ((pos >> 7) << 8) + (pos & 127)
                park = base + _PARK
                av_s[p, pl.ds(j * 16, 16)] = a
                bv_s[p, pl.ds(j * 16, 16)] = b
                fr_s[p, pl.ds(j * 16, 16)] = jnp.where(live, fr, park + lane)
                fc_s[p, pl.ds(j * 16, 16)] = jnp.where(live, fr + 128,
                                                       park + 128 + lane)
                ch_s[p, pl.ds(j * 16, 16)] = jnp.where(
                    live, b, K_PAD + ((k * 128 + j * 16 + lane) & (P_TRASH - 1)))
                off = off + cnt_g
            pltpu.async_copy(av_s.at[p], flat_sh.at[fr_s.at[p]], ssem[p])
            pltpu.async_copy(bv_s.at[p], flat_sh.at[fc_s.at[p]], ssem[p])
            pltpu.async_copy(ones_v.at[0], hist.at[ch_s.at[p]], ssem[p],
                             add=True)

            @pl.when(k + 2 < IDXR)
            def _():
                pltpu.async_copy(rank_hbm.at[ridx_v.at[k + 2]], rr_v.at[p],
                                 gsem[p])
                pltpu.async_copy(rank_hbm.at[cidx_v.at[k + 2]], rc_v.at[p],
                                 gsem[p])

            return off

        def step(kk, off):
            off = chunk(2 * kk, 0, off)
            off = chunk(2 * kk + 1, 1, off)
            return off

        cnt = lax.fori_loop(0, IDXR // 2, step, jnp.int32(0))
        for p in range(2):
            pltpu.make_async_copy(av_s.at[p], flat_sh.at[fr_s.at[p]],
                                  ssem[p]).wait()
            pltpu.make_async_copy(bv_s.at[p], flat_sh.at[fc_s.at[p]],
                                  ssem[p]).wait()
            pltpu.make_async_copy(ones_v.at[0], hist.at[ch_s.at[p]],
                                  ssem[p]).wait()

        nch = jnp.maximum(((cnt + 511) // 512) * 4, 4)

        def fill(cidx, carry):
            pltpu.sync_copy(flat_sh.at[pl.ds(s * (_FLATW + 256) + cidx * 256, 256)], fb_s)
            lane = lax.iota(jnp.int32, 16)
            for j in range(8):
                slot = cidx * 128 + j * 16 + lane
                keep = slot < cnt
                cur_r = fb_s[pl.ds(j * 16, 16)]
                cur_c = fb_s[pl.ds(128 + j * 16, 16)]
                fb_s[pl.ds(j * 16, 16)] = jnp.where(keep, cur_r, slot & 4095)
                fb_s[pl.ds(128 + j * 16, 16)] = jnp.where(
                    keep, cur_c, K_PAD + (slot & (P_TRASH - 1)))
            pltpu.sync_copy(fb_s, flat_sh.at[pl.ds(s * (_FLATW + 256) + cidx * 256, 256)])
            return carry

        lax.fori_loop(cnt >> 7, nch, fill, 0)

        cnt_v[0, pl.ds(0, 16)] = jnp.broadcast_to(nch, (16,))
        pltpu.sync_copy(cnt_v.at[0], cnt_hbm.at[wid])
        pltpu.sync_copy(flat_sh.at[pl.ds(s * (_FLATW + 256), _FLATW)], rc2_hbm.at[wid])
        plsc.subcore_barrier()
        pltpu.sync_copy(hist.at[pl.ds(s * rows_per_sub, rows_per_sub)],
                        hist_hbm.at[c, pl.ds(s * rows_per_sub, rows_per_sub)])

    return body(ridx3, cidx3, rank1, zerosk)


def _sc_scatter_rows(z, rank2):
    n_chunks = N_PAD // 128

    @functools.partial(
        pl.kernel,
        out_type=jax.ShapeDtypeStruct((K_ACC, D), jnp.float32),
        mesh=_MESH,
        scratch_types=[
            pltpu.VMEM((128, D), jnp.float32),
            pltpu.VMEM((1, 128), jnp.int32),
            pltpu.VMEM((1, 128), jnp.int32),
            pltpu.SemaphoreType.DMA,
        ],
    )
    def body(z_hbm, rank_hbm, out_hbm, rows_v, rk_v, idx_v, sem):
        c = lax.axis_index("c")
        s = lax.axis_index("s")
        wid = c * NS + s

        def step(t, carry):
            cid = wid + NW * t

            @pl.when(cid < n_chunks)
            def _():
                pltpu.sync_copy(z_hbm.at[pl.ds(cid * 128, 128)], rows_v)
                pltpu.sync_copy(rank_hbm.at[cid], rk_v.at[0])
                for j in range(8):
                    rk = rk_v[0, pl.ds(j * 16, 16)]
                    nid = cid * 128 + j * 16 + lax.iota(jnp.int32, 16)
                    idx = jnp.where(rk < K, rk, K_PAD + (nid & (P_TRASH - 1)))
                    idx_v[0, pl.ds(j * 16, 16)] = idx
                pltpu.async_copy(rows_v, out_hbm.at[idx_v.at[0]], sem).wait()

            return carry

        lax.fori_loop(0, (n_chunks + NW - 1) // NW, step, 0)

    return body(z, rank2)


_BLK = 512


def _tc_lin1(x, W, b):
    def body(x_ref, w_ref, b_ref, o_ref):
        o_ref[...] = _leaky(
            jnp.dot(x_ref[...], w_ref[...], preferred_element_type=jnp.float32)
            + b_ref[...])

    return pl.pallas_call(
        body,
        grid=(N_PAD // _BLK,),
        in_specs=[
            pl.BlockSpec((_BLK, D), lambda i: (i, 0)),
            pl.BlockSpec((D, D), lambda i: (0, 0)),
            pl.BlockSpec((1, D), lambda i: (0, 0)),
        ],
        out_specs=pl.BlockSpec((_BLK, D), lambda i: (i, 0)),
        out_shape=jax.ShapeDtypeStruct((N_PAD, D), jnp.float32),
    )(x, W, b.reshape(1, D))


def _tc_prep(h, W, d0, d1):
    n = h.shape[0]

    def body(h_ref, w_ref, d0_ref, d1_ref, y_ref, di_ref):
        dinv = lax.rsqrt(d0_ref[...] + d1_ref[...] + 1.0)
        y_ref[...] = dinv * jnp.dot(h_ref[...], w_ref[...],
                                    preferred_element_type=jnp.float32)
        di_ref[...] = dinv

    return pl.pallas_call(
        body,
        grid=(n // _BLK,),
        in_specs=[
            pl.BlockSpec((_BLK, D), lambda i: (i, 0)),
            pl.BlockSpec((D, D), lambda i: (0, 0)),
            pl.BlockSpec((_BLK, 1), lambda i: (i, 0)),
            pl.BlockSpec((_BLK, 1), lambda i: (i, 0)),
        ],
        out_specs=[
            pl.BlockSpec((_BLK, D), lambda i: (i, 0)),
            pl.BlockSpec((_BLK, 1), lambda i: (i, 0)),
        ],
        out_shape=[
            jax.ShapeDtypeStruct((n, D), jnp.float32),
            jax.ShapeDtypeStruct((n, 1), jnp.float32),
        ],
    )(h, W, d0.reshape(n, 1), d1.reshape(n, 1))


def _tc_mid(p0, p1, y_prev, dinv, b_prev, W, scale_out):
    n = y_prev.shape[0]

    def body(p0_ref, p1_ref, y_ref, di_ref, b_ref, w_ref, so_ref, o_ref):
        h = _leaky(di_ref[...] * (p0_ref[...] + p1_ref[...] + y_ref[...])
                   + b_ref[...])
        o_ref[...] = so_ref[...] * jnp.dot(h, w_ref[...],
                                           preferred_element_type=jnp.float32)

    return pl.pallas_call(
        body,
        grid=(n // _BLK,),
        in_specs=[
            pl.BlockSpec((_BLK, D), lambda i: (i, 0)),
            pl.BlockSpec((_BLK, D), lambda i: (i, 0)),
            pl.BlockSpec((_BLK, D), lambda i: (i, 0)),
            pl.BlockSpec((_BLK, 1), lambda i: (i, 0)),
            pl.BlockSpec((1, D), lambda i: (0, 0)),
            pl.BlockSpec((D, D), lambda i: (0, 0)),
            pl.BlockSpec((_BLK, 1), lambda i: (i, 0)),
        ],
        out_specs=pl.BlockSpec((_BLK, D), lambda i: (i, 0)),
        out_shape=jax.ShapeDtypeStruct((n, D), jnp.float32),
    )(p0, p1, y_prev, dinv, b_prev.reshape(1, D), W, scale_out)


def _tc_post(p0, p1, y_prev, dinv, b_prev):
    n = y_prev.shape[0]

    def body(p0_ref, p1_ref, y_ref, di_ref, b_ref, o_ref):
        o_ref[...] = _leaky(
            di_ref[...] * (p0_ref[...] + p1_ref[...] + y_ref[...]) + b_ref[...])

    return pl.pallas_call(
        body,
        grid=(n // _BLK,),
        in_specs=[
            pl.BlockSpec((_BLK, D), lambda i: (i, 0)),
            pl.BlockSpec((_BLK, D), lambda i: (i, 0)),
            pl.BlockSpec((_BLK, D), lambda i: (i, 0)),
            pl.BlockSpec((_BLK, 1), lambda i: (i, 0)),
            pl.BlockSpec((1, D), lambda i: (0, 0)),
        ],
        out_specs=pl.BlockSpec((_BLK, D), lambda i: (i, 0)),
        out_shape=jax.ShapeDtypeStruct((n, D), jnp.float32),
    )(p0, p1, y_prev, dinv, b_prev.reshape(1, D))


def _tc_score(pp0, pp1, h2, Wrel_p, Wroot_p, brel):
    def body(p0_ref, p1_ref, h_ref, wr_ref, wo_ref, b_ref, o_ref):
        aggr = p0_ref[...] + p1_ref[...]
        sc = (jnp.dot(aggr, wr_ref[...], preferred_element_type=jnp.float32)
              + jnp.dot(h_ref[...], wo_ref[...], preferred_element_type=jnp.float32)
              + b_ref[...])
        o_ref[...] = jnp.tanh(sc)

    return pl.pallas_call(
        body,
        grid=(N_PAD // _BLK,),
        in_specs=[
            pl.BlockSpec((_BLK, D), lambda i: (i, 0)),
            pl.BlockSpec((_BLK, D), lambda i: (i, 0)),
            pl.BlockSpec((_BLK, D), lambda i: (i, 0)),
            pl.BlockSpec((D, D), lambda i: (0, 0)),
            pl.BlockSpec((D, D), lambda i: (0, 0)),
            pl.BlockSpec((1, 1), lambda i: (0, 0)),
        ],
        out_specs=pl.BlockSpec((_BLK, D), lambda i: (i, 0)),
        out_shape=jax.ShapeDtypeStruct((N_PAD, D), jnp.float32),
    )(pp0, pp1, h2, Wrel_p, Wroot_p, brel.reshape(1, 1))


_JBLK = 2048


def _tc_rank(s_col, s_row):
    def body(sc_ref, sr_ref, o_ref):
        i = pl.program_id(0)
        j = pl.program_id(1)
        si = sc_ref[...]
        sj = sr_ref[...]
        ii = (lax.broadcasted_iota(jnp.int32, (_BLK, _JBLK), 0) + i * _BLK)
        jj = (lax.broadcasted_iota(jnp.int32, (_BLK, _JBLK), 1) + j * _JBLK)
        si_e = jnp.where(ii < N, si, -2.0)
        sj_e = jnp.where(jj < N, sj, -2.0)
        t = jnp.where(sj_e > si_e, 1, 0) + jnp.where(
            (sj_e == si_e) & (jj < ii), 1, 0)
        part = jnp.sum(t, axis=1, keepdims=True)

        @pl.when(j == 0)
        def _():
            o_ref[...] = jnp.zeros_like(o_ref)

        o_ref[...] += part

    return pl.pallas_call(
        body,
        grid=(N_PAD // _BLK, N_PAD // _JBLK),
        in_specs=[
            pl.BlockSpec((_BLK, 1), lambda i, j: (i, 0)),
            pl.BlockSpec((1, _JBLK), lambda i, j: (0, j)),
        ],
        out_specs=pl.BlockSpec((_BLK, 1), lambda i, j: (i, 0)),
        out_shape=jax.ShapeDtypeStruct((N_PAD, 1), jnp.int32),
    )(s_col, s_row)


def _tc_zmul(h2, s_col):
    def body(h_ref, s_ref, o_ref):
        o_ref[...] = h_ref[...] * s_ref[...]

    return pl.pallas_call(
        body,
        grid=(N_PAD // _BLK,),
        in_specs=[
            pl.BlockSpec((_BLK, D), lambda i: (i, 0)),
            pl.BlockSpec((_BLK, 1), lambda i: (i, 0)),
        ],
        out_specs=pl.BlockSpec((_BLK, D), lambda i: (i, 0)),
        out_shape=jax.ShapeDtypeStruct((N_PAD, D), jnp.float32),
    )(h2, s_col)


def _tc_final(p0, p1, y_prev, dinv, b_prev, W2, b2):
    n = y_prev.shape[0]

    def body(p0_ref, p1_ref, y_ref, di_ref, b_ref, w_ref, b2_ref, o_ref):
        h = _leaky(di_ref[...] * (p0_ref[...] + p1_ref[...] + y_ref[...])
                   + b_ref[...])
        o_ref[...] = jnp.maximum(
            jnp.dot(h, w_ref[...], preferred_element_type=jnp.float32)
            + b2_ref[...], 0.0)

    return pl.pallas_call(
        body,
        grid=(n // _BLK,),
        in_specs=[
            pl.BlockSpec((_BLK, D), lambda i: (i, 0)),
            pl.BlockSpec((_BLK, D), lambda i: (i, 0)),
            pl.BlockSpec((_BLK, D), lambda i: (i, 0)),
            pl.BlockSpec((_BLK, 1), lambda i: (i, 0)),
            pl.BlockSpec((1, D), lambda i: (0, 0)),
            pl.BlockSpec((D, D), lambda i: (0, 0)),
            pl.BlockSpec((1, D), lambda i: (0, 0)),
        ],
        out_specs=pl.BlockSpec((_BLK, D), lambda i: (i, 0)),
        out_shape=jax.ShapeDtypeStruct((n, D), jnp.float32),
    )(p0, p1, y_prev, dinv, b_prev.reshape(1, D), W2, b2.reshape(1, D))


def kernel(x, edge_index, lin1_W, lin1_b, conv1_W, conv1_b, conv2_W, conv2_b,
           pool_Wrel, pool_brel, pool_Wroot, conv3_W, conv3_b, conv4_W, conv4_b,
           conv5_W, conv5_b, conv6_W, conv6_b, lin2_W, lin2_b):
    row = edge_index[0]
    col = edge_index[1]

    epad = E_PAD - E
    pad_ids = jnp.arange(epad, dtype=jnp.int32)
    row_p = jnp.concatenate([row, pad_ids % N])
    col_p = jnp.concatenate([col, N + (pad_ids % (N_PAD - N))])
    ridx3 = row_p.reshape(NW, IDXR, 128)
    cidx3 = col_p.reshape(NW, IDXR, 128)
    rcidx4 = jnp.stack([ridx3, cidx3], axis=2)
    counts_full = jnp.full((NW, 16), IDXR, jnp.int32)

    zeros2 = jnp.zeros((N_PAD, D), jnp.float32)
    zeros1 = jnp.zeros((N_PAD,), jnp.float32)
    zeros1k = jnp.zeros((K_ACC,), jnp.float32)

    xp = jnp.zeros((N_PAD, D), jnp.float32).at[:N].set(x)

    degp = _sc_degree(cidx3, zeros1)

    h1 = _tc_lin1(xp, lin1_W, lin1_b)

    y1, dinv1 = _tc_prep(h1, conv1_W, degp[0], degp[1])
    P1 = _sc_segsum(y1, rcidx4, counts_full, zeros2, N_PAD)
    y2 = _tc_mid(P1[0], P1[1], y1, dinv1, conv1_b, conv2_W, dinv1)
    P2 = _sc_segsum(y2, rcidx4, counts_full, zeros2, N_PAD)
    h2 = _tc_post(P2[0], P2[1], y2, dinv1, conv2_b)

    P3 = _sc_segsum(h2, rcidx4, counts_full, zeros2, N_PAD)
    Wrel_p = jnp.zeros((D, D), jnp.float32).at[:, 0:1].set(pool_Wrel)
    Wroot_p = jnp.zeros((D, D), jnp.float32).at[:, 0:1].set(pool_Wroot)
    s_full = _tc_score(P3[0], P3[1], h2, Wrel_p, Wroot_p, pool_brel)
    s_col = s_full[:, 0:1]
    s_row = s_col.reshape(1, N_PAD)
    rank = _tc_rank(s_col, s_row)
    rank1 = rank.reshape(N_PAD)
    rank2 = rank.reshape(IDXR, 128)

    z = _tc_zmul(h2, s_col)
    rc2_flat, counts2, hist2p = _sc_remap(ridx3, cidx3, rank1, zeros1k)
    rcidx4_2 = rc2_flat.reshape(NW, IDXR, 2, 128)
    h_pool = _sc_scatter_rows(z, rank2)

    def padN(y):
        return jnp.zeros((N_PAD, D), jnp.float32).at[:K_PAD].set(y)

    hp = h_pool[:K_PAD]
    d2a = hist2p[0][:K_PAD]
    d2b = hist2p[1][:K_PAD]
    y3, dinv2 = _tc_prep(hp, conv3_W, d2a, d2b)
    P4 = _sc_segsum(padN(y3), rcidx4_2, counts2, zeros2, N_PAD)
    y4 = _tc_mid(P4[0][:K_PAD], P4[1][:K_PAD], y3, dinv2, conv3_b, conv4_W, dinv2)
    P5 = _sc_segsum(padN(y4), rcidx4_2, counts2, zeros2, N_PAD)
    y5 = _tc_mid(P5[0][:K_PAD], P5[1][:K_PAD], y4, dinv2, conv4_b, conv5_W, dinv2)
    P6 = _sc_segsum(padN(y5), rcidx4_2, counts2, zeros2, N_PAD)
    y6 = _tc_mid(P6[0][:K_PAD], P6[1][:K_PAD], y5, dinv2, conv5_b, conv6_W, dinv2)
    P7 = _sc_segsum(padN(y6), rcidx4_2, counts2, zeros2, N_PAD)
    out = _tc_final(P7[0][:K_PAD], P7[1][:K_PAD], y6, dinv2, conv6_b,
                    lin2_W, lin2_b)
    return out[:K]


# speedup vs baseline: 25.1725x; 1.0440x over previous
import functools

import jax
import jax.numpy as jnp
from jax import lax
from jax.experimental import pallas as pl
from jax.experimental.pallas import tpu as pltpu
from jax.experimental.pallas import tpu_sc as plsc

N = 10000
E = 320000
D = 128
K = 5000

NC = 2
NS = 16
NW = NC * NS

N_PAD = 10240
EPW = 10240
E_PAD = NW * EPW
IDXR = EPW // 128

K_PAD = 5120
P_TRASH = 1024
K_ACC = K_PAD + P_TRASH

_MESH = plsc.VectorSubcoreMesh(core_axis_name="c", subcore_axis_name="s")


def _leaky(v):
    return jnp.where(v > 0, v, 0.01 * v)


def _sc_degree(cidx3, zeros1):
    @functools.partial(
        pl.kernel,
        out_type=jax.ShapeDtypeStruct((NC, N_PAD), jnp.float32),
        mesh=_MESH,
        scratch_types=[
            pltpu.VMEM((IDXR, 128), jnp.int32),
            pltpu.VMEM((1, 128), jnp.float32),
            pltpu.VMEM_SHARED((N_PAD,), jnp.float32),
            pltpu.SemaphoreType.DMA,
        ],
    )
    def body(cidx_hbm, zeros_hbm, out_hbm, cidx_v, ones_v, hist, sem):
        c = lax.axis_index("c")
        s = lax.axis_index("s")
        wid = c * NS + s
        rows_per_sub = N_PAD // NS
        pltpu.sync_copy(zeros_hbm.at[pl.ds(s * rows_per_sub, rows_per_sub)],
                        hist.at[pl.ds(s * rows_per_sub, rows_per_sub)])
        pltpu.sync_copy(cidx_hbm.at[wid], cidx_v)
        for j in range(8):
            ones_v[0, pl.ds(j * 16, 16)] = jnp.ones((16,), jnp.float32)
        plsc.subcore_barrier()

        def step(k, carry):
            pltpu.sync_copy(ones_v.at[0], hist.at[cidx_v.at[k]], add=True)
            return carry

        lax.fori_loop(0, IDXR, step, 0)
        plsc.subcore_barrier()
        pltpu.sync_copy(hist.at[pl.ds(s * rows_per_sub, rows_per_sub)],
                        out_hbm.at[c, pl.ds(s * rows_per_sub, rows_per_sub)])

    return body(cidx3, zeros1)


def _sc_segsum(y, rcidx4, counts, zeros2, n_acc):
    @functools.partial(
        pl.kernel,
        out_type=jax.ShapeDtypeStruct((NC, n_acc, D), jnp.float32),
        mesh=_MESH,
        scratch_types=[
            [pltpu.VMEM((2, 128), jnp.int32) for _ in range(4)],
            pltpu.VMEM((128, D), jnp.float32),
            pltpu.VMEM((128, D), jnp.float32),
            pltpu.VMEM((1, 16), jnp.int32),
            pltpu.VMEM_SHARED((n_acc, D), jnp.float32),
            [pltpu.SemaphoreType.DMA for _ in range(4)],
            [pltpu.SemaphoreType.DMA for _ in range(2)],
            [pltpu.SemaphoreType.DMA for _ in range(2)],
        ],
    )
    def body(y_hbm, rc_hbm, cnt_hbm, zeros_hbm, out_hbm,
             idx, buf0, buf1, cnt_v, acc, isem, gsem, ssem):
        c = lax.axis_index("c")
        s = lax.axis_index("s")
        wid = c * NS + s
        buf = (buf0, buf1)
        rows_per_sub = n_acc // NS
        pltpu.sync_copy(cnt_hbm.at[wid], cnt_v.at[0])
        pltpu.sync_copy(zeros_hbm.at[pl.ds(s * rows_per_sub, rows_per_sub)],
                        acc.at[pl.ds(s * rows_per_sub, rows_per_sub)])
        nch = cnt_v[0, pl.ds(0, 16)][0]
        plsc.subcore_barrier()

        pltpu.async_copy(rc_hbm.at[wid, 0], idx[0], isem[0])
        pltpu.async_copy(rc_hbm.at[wid, 1], idx[1], isem[1])

        def phase(k, K, kk):
            b = K % 2
            pb = (K - 1) % 2
            pq = (K - 1) % 4
            nq = (K + 2) % 4
            pltpu.make_async_copy(rc_hbm.at[wid, k], idx[K], isem[K]).wait()

            @pl.when(k >= 2)
            def _():
                pltpu.make_async_copy(buf[b], acc.at[idx[nq].at[1]], ssem[b]).wait()

            pltpu.async_copy(y_hbm.at[idx[K].at[0]], buf[b], gsem[b])

            @pl.when(k >= 1)
            def _():
                pltpu.make_async_copy(y_hbm.at[idx[pq].at[0]], buf[pb],
                                      gsem[pb]).wait()
                pltpu.async_copy(buf[pb], acc.at[idx[pq].at[1]], ssem[pb],
                                 add=True)

            @pl.when(k + 2 < nch)
            def _():
                pltpu.async_copy(rc_hbm.at[wid, k + 2], idx[nq], isem[nq])

        def step(kk, carry):
            for K in range(4):
                phase(4 * kk + K, K, kk)
            return carry

        lax.fori_loop(0, nch // 4, step, 0)
        pltpu.make_async_copy(y_hbm.at[idx[3].at[0]], buf1, gsem[1]).wait()
        pltpu.sync_copy(buf1, acc.at[idx[3].at[1]], add=True)
        pltpu.make_async_copy(buf0, acc.at[idx[2].at[1]], ssem[0]).wait()
        plsc.subcore_barrier()
        pltpu.sync_copy(acc.at[pl.ds(s * rows_per_sub, rows_per_sub)],
                        out_hbm.at[c, pl.ds(s * rows_per_sub, rows_per_sub)])

    return body(y, rcidx4, counts, zeros2)


_FLATW = IDXR * 2 * 128
_PARK = _FLATW


def _sc_remap(ridx3, cidx3, rank1, zerosk):
    @functools.partial(
        pl.kernel,
        out_type=(
            jax.ShapeDtypeStruct((NW, _FLATW), jnp.int32),
            jax.ShapeDtypeStruct((NW, 16), jnp.int32),
            jax.ShapeDtypeStruct((NC, K_ACC), jnp.float32),
        ),
        mesh=_MESH,
        scratch_types=[
            pltpu.VMEM((IDXR, 128), jnp.int32),
            pltpu.VMEM((IDXR, 128), jnp.int32),
            pltpu.VMEM((2, 128), jnp.int32),
            pltpu.VMEM((2, 128), jnp.int32),
            pltpu.VMEM((2, 128), jnp.int32),
            pltpu.VMEM((2, 128), jnp.int32),
            pltpu.VMEM((2, 128), jnp.int32),
            pltpu.VMEM((2, 128), jnp.int32),
            pltpu.VMEM((2, 128), jnp.int32),
            pltpu.VMEM((256,), jnp.int32),
            pltpu.VMEM((1, 128), jnp.float32),
            pltpu.VMEM((1, 16), jnp.int32),
            pltpu.VMEM((1, 48), jnp.int32),
            pltpu.VMEM_SHARED((NS * (_FLATW + 256),), jnp.int32),
            pltpu.VMEM_SHARED((K_ACC,), jnp.float32),
            [pltpu.SemaphoreType.DMA for _ in range(2)],
            [pltpu.SemaphoreType.DMA for _ in range(2)],
        ],
    )
    def body(ridx_hbm, cidx_hbm, rank_hbm, zeros_hbm, rc2_hbm, cnt_hbm,
             hist_hbm, ridx_v, cidx_v, rr_v, rc_v, av_s, bv_s, fr_s, fc_s,
             ch_s, fb_s, ones_v, cnt_v, scr, flat_sh, hist, gsem, ssem):
        c = lax.axis_index("c")
        s = lax.axis_index("s")
        wid = c * NS + s
        rows_per_sub = K_ACC // NS
        pltpu.sync_copy(zeros_hbm.at[pl.ds(s * rows_per_sub, rows_per_sub)],
                        hist.at[pl.ds(s * rows_per_sub, rows_per_sub)])
        pltpu.sync_copy(ridx_hbm.at[wid], ridx_v)
        pltpu.sync_copy(cidx_hbm.at[wid], cidx_v)
        for j in range(8):
            ones_v[0, pl.ds(j * 16, 16)] = jnp.ones((16,), jnp.float32)
        scr[0, pl.ds(0, 16)] = jnp.zeros((16,), jnp.int32)
        plsc.subcore_barrier()

        pltpu.async_copy(rank_hbm.at[ridx_v.at[0]], rr_v.at[0], gsem[0])
        pltpu.async_copy(rank_hbm.at[cidx_v.at[0]], rc_v.at[0], gsem[0])
        pltpu.async_copy(rank_hbm.at[ridx_v.at[1]], rr_v.at[1], gsem[1])
        pltpu.async_copy(rank_hbm.at[cidx_v.at[1]], rc_v.at[1], gsem[1])

        def chunk(k, p, off):
            pltpu.make_async_copy(rank_hbm.at[ridx_v.at[k]], rr_v.at[p],
                                  gsem[p]).wait()
            pltpu.make_async_copy(rank_hbm.at[cidx_v.at[k]], rc_v.at[p],
                                  gsem[p]).wait()

            @pl.when(k >= 2)
            def _():
                pltpu.make_async_copy(av_s.at[p], flat_sh.at[fr_s.at[p]],
                                      ssem[p]).wait()
                pltpu.make_async_copy(bv_s.at[p], flat_sh.at[fc_s.at[p]],
                                      ssem[p]).wait()
                pltpu.make_async_copy(ones_v.at[0], hist.at[ch_s.at[p]],
                                      ssem[p]).wait()

            lane = lax.iota(jnp.int32, 16)
            for j in range(8):
                a = rr_v[p, pl.ds(j * 16, 16)]
                b = rc_v[p, pl.ds(j * 16, 16)]
                live = (a < K) & (b < K)
                liveint = jnp.where(live, 1, 0)
                cum = liveint
                for d in (1, 2, 4, 8):
                    scr[0, pl.ds(16, 16)] = cum
                    cum = cum + scr[0, pl.ds(16 - d, 16)]
                scr[0, pl.ds(16, 16)] = cum
                cnt_g = scr[0, pl.ds(31, 16)][0]
                pos = off + cum - liveint
                base = s * (_FLATW + 256)
                fr = base + ((pos >> 7) << 8) + (pos & 127)
                park = base + _PARK
                av_s[p, pl.ds(j * 16, 16)] = a
                bv_s[p, pl.ds(j * 16, 16)] = b
                fr_s[p, pl.ds(j * 16, 16)] = jnp.where(live, fr, park + lane)
                fc_s[p, pl.ds(j * 16, 16)] = jnp.where(live, fr + 128,
                                                       park + 128 + lane)
                ch_s[p, pl.ds(j * 16, 16)] = jnp.where(
                    live, b, K_PAD + ((k * 128 + j * 16 + lane) & (P_TRASH - 1)))
                off = off + cnt_g
            pltpu.async_copy(av_s.at[p], flat_sh.at[fr_s.at[p]], ssem[p])
            pltpu.async_copy(bv_s.at[p], flat_sh.at[fc_s.at[p]], ssem[p])
            pltpu.async_copy(ones_v.at[0], hist.at[ch_s.at[p]], ssem[p],
                             add=True)

            @pl.when(k + 2 < IDXR)
            def _():
                pltpu.async_copy(rank_hbm.at[ridx_v.at[k + 2]], rr_v.at[p],
                                 gsem[p])
                pltpu.async_copy(rank_hbm.at[cidx_v.at[k + 2]], rc_v.at[p],
                                 gsem[p])

            return off

        def step(kk, off):
            off = chunk(2 * kk, 0, off)
            off = chunk(2 * kk + 1, 1, off)
            return off

        cnt = lax.fori_loop(0, IDXR // 2, step, jnp.int32(0))
        for p in range(2):
            pltpu.make_async_copy(av_s.at[p], flat_sh.at[fr_s.at[p]],
                                  ssem[p]).wait()
            pltpu.make_async_copy(bv_s.at[p], flat_sh.at[fc_s.at[p]],
                                  ssem[p]).wait()
            pltpu.make_async_copy(ones_v.at[0], hist.at[ch_s.at[p]],
                                  ssem[p]).wait()

        nch = jnp.maximum(((cnt + 511) // 512) * 4, 4)

        def fill(cidx, carry):
            pltpu.sync_copy(flat_sh.at[pl.ds(s * (_FLATW + 256) + cidx * 256, 256)], fb_s)
            lane = lax.iota(jnp.int32, 16)
            for j in range(8):
                slot = cidx * 128 + j * 16 + lane
                keep = slot < cnt
                cur_r = fb_s[pl.ds(j * 16, 16)]
                cur_c = fb_s[pl.ds(128 + j * 16, 16)]
                fb_s[pl.ds(j * 16, 16)] = jnp.where(keep, cur_r, slot & 4095)
                fb_s[pl.ds(128 + j * 16, 16)] = jnp.where(
                    keep, cur_c, K_PAD + (slot & (P_TRASH - 1)))
            pltpu.sync_copy(fb_s, flat_sh.at[pl.ds(s * (_FLATW + 256) + cidx * 256, 256)])
            return carry

        lax.fori_loop(cnt >> 7, nch, fill, 0)

        cnt_v[0, pl.ds(0, 16)] = jnp.broadcast_to(nch, (16,))
        pltpu.sync_copy(cnt_v.at[0], cnt_hbm.at[wid])
        pltpu.sync_copy(flat_sh.at[pl.ds(s * (_FLATW + 256), _FLATW)], rc2_hbm.at[wid])
        plsc.subcore_barrier()
        pltpu.sync_copy(hist.at[pl.ds(s * rows_per_sub, rows_per_sub)],
                        hist_hbm.at[c, pl.ds(s * rows_per_sub, rows_per_sub)])

    return body(ridx3, cidx3, rank1, zerosk)


def _sc_scatter_rows(z, rank2):
    n_chunks = N_PAD // 128

    @functools.partial(
        pl.kernel,
        out_type=jax.ShapeDtypeStruct((K_ACC, D), jnp.float32),
        mesh=_MESH,
        scratch_types=[
            pltpu.VMEM((128, D), jnp.float32),
            pltpu.VMEM((1, 128), jnp.int32),
            pltpu.VMEM((1, 128), jnp.int32),
            pltpu.SemaphoreType.DMA,
        ],
    )
    def body(z_hbm, rank_hbm, out_hbm, rows_v, rk_v, idx_v, sem):
        c = lax.axis_index("c")
        s = lax.axis_index("s")
        wid = c * NS + s

        def step(t, carry):
            cid = wid + NW * t

            @pl.when(cid < n_chunks)
            def _():
                pltpu.sync_copy(z_hbm.at[pl.ds(cid * 128, 128)], rows_v)
                pltpu.sync_copy(rank_hbm.at[cid], rk_v.at[0])
                for j in range(8):
                    rk = rk_v[0, pl.ds(j * 16, 16)]
                    nid = cid * 128 + j * 16 + lax.iota(jnp.int32, 16)
                    idx = jnp.where(rk < K, rk, K_PAD + (nid & (P_TRASH - 1)))
                    idx_v[0, pl.ds(j * 16, 16)] = idx
                pltpu.async_copy(rows_v, out_hbm.at[idx_v.at[0]], sem).wait()

            return carry

        lax.fori_loop(0, (n_chunks + NW - 1) // NW, step, 0)

    return body(z, rank2)


_BLK = 512


def _tc_lin1_prep(x, Wl, bl, W1, d0, d1):
    def body(x_ref, wl_ref, bl_ref, w1_ref, d0_ref, d1_ref, y_ref, di_ref):
        h = _leaky(
            jnp.dot(x_ref[...], wl_ref[...], preferred_element_type=jnp.float32)
            + bl_ref[...])
        dinv = lax.rsqrt(d0_ref[...] + d1_ref[...] + 1.0)
        y_ref[...] = dinv * jnp.dot(h, w1_ref[...],
                                    preferred_element_type=jnp.float32)
        di_ref[...] = dinv

    return pl.pallas_call(
        body,
        grid=(N_PAD // _BLK,),
        in_specs=[
            pl.BlockSpec((_BLK, D), lambda i: (i, 0)),
            pl.BlockSpec((D, D), lambda i: (0, 0)),
            pl.BlockSpec((1, D), lambda i: (0, 0)),
            pl.BlockSpec((D, D), lambda i: (0, 0)),
            pl.BlockSpec((_BLK, 1), lambda i: (i, 0)),
            pl.BlockSpec((_BLK, 1), lambda i: (i, 0)),
        ],
        out_specs=[
            pl.BlockSpec((_BLK, D), lambda i: (i, 0)),
            pl.BlockSpec((_BLK, 1), lambda i: (i, 0)),
        ],
        out_shape=[
            jax.ShapeDtypeStruct((N_PAD, D), jnp.float32),
            jax.ShapeDtypeStruct((N_PAD, 1), jnp.float32),
        ],
    )(x, Wl, bl.reshape(1, D), W1, d0.reshape(N_PAD, 1), d1.reshape(N_PAD, 1))


def _tc_prep(h, W, d0, d1):
    n = h.shape[0]

    def body(h_ref, w_ref, d0_ref, d1_ref, y_ref, di_ref):
        dinv = lax.rsqrt(d0_ref[...] + d1_ref[...] + 1.0)
        y_ref[...] = dinv * jnp.dot(h_ref[...], w_ref[...],
                                    preferred_element_type=jnp.float32)
        di_ref[...] = dinv

    return pl.pallas_call(
        body,
        grid=(n // _BLK,),
        in_specs=[
            pl.BlockSpec((_BLK, D), lambda i: (i, 0)),
            pl.BlockSpec((D, D), lambda i: (0, 0)),
            pl.BlockSpec((_BLK, 1), lambda i: (i, 0)),
            pl.BlockSpec((_BLK, 1), lambda i: (i, 0)),
        ],
        out_specs=[
            pl.BlockSpec((_BLK, D), lambda i: (i, 0)),
            pl.BlockSpec((_BLK, 1), lambda i: (i, 0)),
        ],
        out_shape=[
            jax.ShapeDtypeStruct((n, D), jnp.float32),
            jax.ShapeDtypeStruct((n, 1), jnp.float32),
        ],
    )(h, W, d0.reshape(n, 1), d1.reshape(n, 1))


def _tc_mid(p0, p1, y_prev, dinv, b_prev, W, scale_out):
    n = y_prev.shape[0]

    def body(p0_ref, p1_ref, y_ref, di_ref, b_ref, w_ref, so_ref, o_ref):
        h = _leaky(di_ref[...] * (p0_ref[...] + p1_ref[...] + y_ref[...])
                   + b_ref[...])
        o_ref[...] = so_ref[...] * jnp.dot(h, w_ref[...],
                                           preferred_element_type=jnp.float32)

    return pl.pallas_call(
        body,
        grid=(n // _BLK,),
        in_specs=[
            pl.BlockSpec((_BLK, D), lambda i: (i, 0)),
            pl.BlockSpec((_BLK, D), lambda i: (i, 0)),
            pl.BlockSpec((_BLK, D), lambda i: (i, 0)),
            pl.BlockSpec((_BLK, 1), lambda i: (i, 0)),
            pl.BlockSpec((1, D), lambda i: (0, 0)),
            pl.BlockSpec((D, D), lambda i: (0, 0)),
            pl.BlockSpec((_BLK, 1), lambda i: (i, 0)),
        ],
        out_specs=pl.BlockSpec((_BLK, D), lambda i: (i, 0)),
        out_shape=jax.ShapeDtypeStruct((n, D), jnp.float32),
    )(p0, p1, y_prev, dinv, b_prev.reshape(1, D), W, scale_out)


def _tc_post(p0, p1, y_prev, dinv, b_prev):
    n = y_prev.shape[0]

    def body(p0_ref, p1_ref, y_ref, di_ref, b_ref, o_ref):
        o_ref[...] = _leaky(
            di_ref[...] * (p0_ref[...] + p1_ref[...] + y_ref[...]) + b_ref[...])

    return pl.pallas_call(
        body,
        grid=(n // _BLK,),
        in_specs=[
            pl.BlockSpec((_BLK, D), lambda i: (i, 0)),
            pl.BlockSpec((_BLK, D), lambda i: (i, 0)),
            pl.BlockSpec((_BLK, D), lambda i: (i, 0)),
            pl.BlockSpec((_BLK, 1), lambda i: (i, 0)),
            pl.BlockSpec((1, D), lambda i: (0, 0)),
        ],
        out_specs=pl.BlockSpec((_BLK, D), lambda i: (i, 0)),
        out_shape=jax.ShapeDtypeStruct((n, D), jnp.float32),
    )(p0, p1, y_prev, dinv, b_prev.reshape(1, D))


def _tc_score(pp0, pp1, h2, Wrel_p, Wroot_p, brel):
    def body(p0_ref, p1_ref, h_ref, wr_ref, wo_ref, b_ref, o_ref, z_ref):
        aggr = p0_ref[...] + p1_ref[...]
        sc = (jnp.dot(aggr, wr_ref[...], preferred_element_type=jnp.float32)
              + jnp.dot(h_ref[...], wo_ref[...], preferred_element_type=jnp.float32)
              + b_ref[...])
        s2d = jnp.tanh(sc)
        o_ref[...] = s2d
        z_ref[...] = h_ref[...] * lax.slice(s2d, (0, 0), (_BLK, 1))

    return pl.pallas_call(
        body,
        grid=(N_PAD // _BLK,),
        in_specs=[
            pl.BlockSpec((_BLK, D), lambda i: (i, 0)),
            pl.BlockSpec((_BLK, D), lambda i: (i, 0)),
            pl.BlockSpec((_BLK, D), lambda i: (i, 0)),
            pl.BlockSpec((D, D), lambda i: (0, 0)),
            pl.BlockSpec((D, D), lambda i: (0, 0)),
            pl.BlockSpec((1, 1), lambda i: (0, 0)),
        ],
        out_specs=[
            pl.BlockSpec((_BLK, D), lambda i: (i, 0)),
            pl.BlockSpec((_BLK, D), lambda i: (i, 0)),
        ],
        out_shape=[
            jax.ShapeDtypeStruct((N_PAD, D), jnp.float32),
            jax.ShapeDtypeStruct((N_PAD, D), jnp.float32),
        ],
    )(pp0, pp1, h2, Wrel_p, Wroot_p, brel.reshape(1, 1))


_JBLK = 2048


def _tc_rank(s_col, s_row):
    def body(sc_ref, sr_ref, o_ref):
        i = pl.program_id(0)
        j = pl.program_id(1)
        si = sc_ref[...]
        sj = sr_ref[...]

        @pl.when(j == 0)
        def _():
            o_ref[...] = jnp.zeros_like(o_ref)

        pure_left = (j + 1) * _JBLK <= i * _BLK
        pure_right = j * _JBLK >= (i + 1) * _BLK

        @pl.when(pure_left)
        def _():
            t = jnp.where(sj >= si, 1, 0)
            o_ref[...] += jnp.sum(t, axis=1, keepdims=True)

        @pl.when(pure_right)
        def _():
            jj = (lax.broadcasted_iota(jnp.int32, (_BLK, _JBLK), 1) + j * _JBLK)
            sj_e = jnp.where(jj < N, sj, -2.0)
            t = jnp.where(sj_e > si, 1, 0)
            o_ref[...] += jnp.sum(t, axis=1, keepdims=True)

        @pl.when(jnp.logical_not(pure_left | pure_right))
        def _():
            ii = (lax.broadcasted_iota(jnp.int32, (_BLK, _JBLK), 0) + i * _BLK)
            jj = (lax.broadcasted_iota(jnp.int32, (_BLK, _JBLK), 1) + j * _JBLK)
            si_e = jnp.where(ii < N, si, -2.0)
            sj_e = jnp.where(jj < N, sj, -2.0)
            t = jnp.where(sj_e > si_e, 1, 0) + jnp.where(
                (sj_e == si_e) & (jj < ii), 1, 0)
            o_ref[...] += jnp.sum(t, axis=1, keepdims=True)

    return pl.pallas_call(
        body,
        grid=(N_PAD // _BLK, N_PAD // _JBLK),
        in_specs=[
            pl.BlockSpec((_BLK, 1), lambda i, j: (i, 0)),
            pl.BlockSpec((1, _JBLK), lambda i, j: (0, j)),
        ],
        out_specs=pl.BlockSpec((_BLK, 1), lambda i, j: (i, 0)),
        out_shape=jax.ShapeDtypeStruct((N_PAD, 1), jnp.int32),
    )(s_col, s_row)


def _tc_zmul(h2, s_col):
    def body(h_ref, s_ref, o_ref):
        o_ref[...] = h_ref[...] * s_ref[...]

    return pl.pallas_call(
        body,
        grid=(N_PAD // _BLK,),
        in_specs=[
            pl.BlockSpec((_BLK, D), lambda i: (i, 0)),
            pl.BlockSpec((_BLK, 1), lambda i: (i, 0)),
        ],
        out_specs=pl.BlockSpec((_BLK, D), lambda i: (i, 0)),
        out_shape=jax.ShapeDtypeStruct((N_PAD, D), jnp.float32),
    )(h2, s_col)


def _tc_final(p0, p1, y_prev, dinv, b_prev, W2, b2):
    n = y_prev.shape[0]

    def body(p0_ref, p1_ref, y_ref, di_ref, b_ref, w_ref, b2_ref, o_ref):
        h = _leaky(di_ref[...] * (p0_ref[...] + p1_ref[...] + y_ref[...])
                   + b_ref[...])
        o_ref[...] = jnp.maximum(
            jnp.dot(h, w_ref[...], preferred_element_type=jnp.float32)
            + b2_ref[...], 0.0)

    return pl.pallas_call(
        body,
        grid=(n // _BLK,),
        in_specs=[
            pl.BlockSpec((_BLK, D), lambda i: (i, 0)),
            pl.BlockSpec((_BLK, D), lambda i: (i, 0)),
            pl.BlockSpec((_BLK, D), lambda i: (i, 0)),
            pl.BlockSpec((_BLK, 1), lambda i: (i, 0)),
            pl.BlockSpec((1, D), lambda i: (0, 0)),
            pl.BlockSpec((D, D), lambda i: (0, 0)),
            pl.BlockSpec((1, D), lambda i: (0, 0)),
        ],
        out_specs=pl.BlockSpec((_BLK, D), lambda i: (i, 0)),
        out_shape=jax.ShapeDtypeStruct((n, D), jnp.float32),
    )(p0, p1, y_prev, dinv, b_prev.reshape(1, D), W2, b2.reshape(1, D))


def kernel(x, edge_index, lin1_W, lin1_b, conv1_W, conv1_b, conv2_W, conv2_b,
           pool_Wrel, pool_brel, pool_Wroot, conv3_W, conv3_b, conv4_W, conv4_b,
           conv5_W, conv5_b, conv6_W, conv6_b, lin2_W, lin2_b):
    row = edge_index[0]
    col = edge_index[1]

    epad = E_PAD - E
    pad_ids = jnp.arange(epad, dtype=jnp.int32)
    row_p = jnp.concatenate([row, pad_ids % N])
    col_p = jnp.concatenate([col, N + (pad_ids % (N_PAD - N))])
    ridx3 = row_p.reshape(NW, IDXR, 128)
    cidx3 = col_p.reshape(NW, IDXR, 128)
    rcidx4 = jnp.stack([ridx3, cidx3], axis=2)
    counts_full = jnp.full((NW, 16), IDXR, jnp.int32)

    zeros2 = jnp.zeros((N_PAD, D), jnp.float32)
    zeros1 = jnp.zeros((N_PAD,), jnp.float32)
    zeros1k = jnp.zeros((K_ACC,), jnp.float32)

    xp = jnp.zeros((N_PAD, D), jnp.float32).at[:N].set(x)

    degp = _sc_degree(cidx3, zeros1)

    y1, dinv1 = _tc_lin1_prep(xp, lin1_W, lin1_b, conv1_W, degp[0], degp[1])
    P1 = _sc_segsum(y1, rcidx4, counts_full, zeros2, N_PAD)
    y2 = _tc_mid(P1[0], P1[1], y1, dinv1, conv1_b, conv2_W, dinv1)
    P2 = _sc_segsum(y2, rcidx4, counts_full, zeros2, N_PAD)
    h2 = _tc_post(P2[0], P2[1], y2, dinv1, conv2_b)

    P3 = _sc_segsum(h2, rcidx4, counts_full, zeros2, N_PAD)
    Wrel_p = jnp.zeros((D, D), jnp.float32).at[:, 0:1].set(pool_Wrel)
    Wroot_p = jnp.zeros((D, D), jnp.float32).at[:, 0:1].set(pool_Wroot)
    s_full, z = _tc_score(P3[0], P3[1], h2, Wrel_p, Wroot_p, pool_brel)
    s_col = s_full[:, 0:1]
    s_row = s_col.reshape(1, N_PAD)
    rank = _tc_rank(s_col, s_row)
    rank1 = rank.reshape(N_PAD)
    rank2 = rank.reshape(IDXR, 128)

    rc2_flat, counts2, hist2p = _sc_remap(ridx3, cidx3, rank1, zeros1k)
    rcidx4_2 = rc2_flat.reshape(NW, IDXR, 2, 128)
    h_pool = _sc_scatter_rows(z, rank2)

    def padN(y):
        return jnp.zeros((N_PAD, D), jnp.float32).at[:K_PAD].set(y)

    hp = h_pool[:K_PAD]
    d2a = hist2p[0][:K_PAD]
    d2b = hist2p[1][:K_PAD]
    y3, dinv2 = _tc_prep(hp, conv3_W, d2a, d2b)
    P4 = _sc_segsum(padN(y3), rcidx4_2, counts2, zeros2, N_PAD)
    y4 = _tc_mid(P4[0][:K_PAD], P4[1][:K_PAD], y3, dinv2, conv3_b, conv4_W, dinv2)
    P5 = _sc_segsum(padN(y4), rcidx4_2, counts2, zeros2, N_PAD)
    y5 = _tc_mid(P5[0][:K_PAD], P5[1][:K_PAD], y4, dinv2, conv4_b, conv5_W, dinv2)
    P6 = _sc_segsum(padN(y5), rcidx4_2, counts2, zeros2, N_PAD)
    y6 = _tc_mid(P6[0][:K_PAD], P6[1][:K_PAD], y5, dinv2, conv5_b, conv6_W, dinv2)
    P7 = _sc_segsum(padN(y6), rcidx4_2, counts2, zeros2, N_PAD)
    out = _tc_final(P7[0][:K_PAD], P7[1][:K_PAD], y6, dinv2, conv6_b,
                    lin2_W, lin2_b)
    return out[:K]
